# Initial kernel scaffold; baseline (speedup 1.0000x reference)
import jax
import jax.numpy as jnp
from jax.experimental import pallas as pl


def kernel(x, edge_index, W1l, b1, W1r, W2l, b2, W2r, W3l, b3, W3r, Wfc, bfc):
    raise NotImplementedError("write your pallas kernel here")


# speedup vs baseline: 7.0814x; 7.0814x over previous
import functools

import jax
import jax.numpy as jnp
from jax import lax
from jax.experimental import pallas as pl
from jax.experimental.pallas import tpu as pltpu
from jax.experimental.pallas import tpu_sc as plsc

N_NODES = 10000
N_PAD = 10240
D = 128
STREAM = 128
N_SC = 2
N_SUB = 16
N_TILES = N_SC * N_SUB
IDX_CHUNK = 8
BLK = 256


def _zero_buf(buf, rows, cols):
  def zrow(i, _):
    def zcol(j, _):
      buf[i, pl.ds(j * 16, 16)] = jnp.zeros((16,), jnp.float32)
      return 0
    lax.fori_loop(0, cols // 16, zcol, 0)
    return 0
  lax.fori_loop(0, rows, zrow, 0)


def _sc_aggregate(table, src2d, dst2d, rows_per_tile):
  mesh = plsc.VectorSubcoreMesh(core_axis_name="c", subcore_axis_name="s")
  per = N_PAD // N_SUB

  @functools.partial(
      pl.kernel,
      out_type=jax.ShapeDtypeStruct((N_SC, N_PAD, D), jnp.float32),
      mesh=mesh,
      scratch_types=[
          pltpu.VMEM((IDX_CHUNK, STREAM), jnp.int32),
          pltpu.VMEM((IDX_CHUNK, STREAM), jnp.int32),
          pltpu.VMEM((STREAM, D), jnp.float32),
          pltpu.VMEM_SHARED((N_PAD, D), jnp.float32),
          pltpu.SemaphoreType.DMA,
      ],
  )
  def agg(table_hbm, src_hbm, dst_hbm, sum_out, src_v, dst_v, rows_v, acc_sh,
          sem):
    c = lax.axis_index("c")
    s = lax.axis_index("s")
    wid = c * N_SUB + s

    _zero_buf(rows_v, STREAM, D)
    def zshared(k, _):
      pltpu.sync_copy(rows_v, acc_sh.at[pl.ds(s * per + k * STREAM, STREAM)])
      return 0
    lax.fori_loop(0, per // STREAM, zshared, 0)
    plsc.subcore_barrier()

    row0 = wid * rows_per_tile
    def chunk_body(k, _):
      base = row0 + k * IDX_CHUNK
      pltpu.sync_copy(src_hbm.at[pl.ds(base, IDX_CHUNK)], src_v)
      pltpu.sync_copy(dst_hbm.at[pl.ds(base, IDX_CHUNK)], dst_v)
      for g in range(IDX_CHUNK):
        pltpu.async_copy(table_hbm.at[src_v.at[g]], rows_v, sem).wait()
        pltpu.sync_copy(rows_v, acc_sh.at[dst_v.at[g]], add=True)
      return 0
    lax.fori_loop(0, rows_per_tile // IDX_CHUNK, chunk_body, 0)
    plsc.subcore_barrier()

    def wb(k, _):
      base = s * per + k * STREAM
      pltpu.sync_copy(acc_sh.at[pl.ds(base, STREAM)], rows_v)
      pltpu.sync_copy(rows_v, sum_out.at[c, pl.ds(base, STREAM)])
      return 0
    lax.fori_loop(0, per // STREAM, wb, 0)

  return agg(table, src2d, dst2d)


def _sc_degree(dst2d, rows_per_tile):
  mesh = plsc.VectorSubcoreMesh(core_axis_name="c", subcore_axis_name="s")
  per = N_PAD // N_SUB

  @functools.partial(
      pl.kernel,
      out_type=jax.ShapeDtypeStruct((N_SC, N_PAD, D), jnp.float32),
      mesh=mesh,
      scratch_types=[
          pltpu.VMEM((IDX_CHUNK, STREAM), jnp.int32),
          pltpu.VMEM((STREAM, D), jnp.float32),
          pltpu.VMEM_SHARED((N_PAD, D), jnp.float32),
      ],
  )
  def deg(dst_hbm, deg_out, dst_v, ones_v, acc_sh):
    c = lax.axis_index("c")
    s = lax.axis_index("s")
    wid = c * N_SUB + s

    _zero_buf(ones_v, STREAM, D)
    def zshared(k, _):
      pltpu.sync_copy(ones_v, acc_sh.at[pl.ds(s * per + k * STREAM, STREAM)])
      return 0
    lax.fori_loop(0, per // STREAM, zshared, 0)

    def orow(i, _):
      def ocol(j, _):
        ones_v[i, pl.ds(j * 16, 16)] = jnp.ones((16,), jnp.float32)
        return 0
      lax.fori_loop(0, D // 16, ocol, 0)
      return 0
    lax.fori_loop(0, STREAM, orow, 0)
    plsc.subcore_barrier()

    row0 = wid * rows_per_tile
    def chunk_body(k, _):
      base = row0 + k * IDX_CHUNK
      pltpu.sync_copy(dst_hbm.at[pl.ds(base, IDX_CHUNK)], dst_v)
      for g in range(IDX_CHUNK):
        pltpu.sync_copy(ones_v, acc_sh.at[dst_v.at[g]], add=True)
      return 0
    lax.fori_loop(0, rows_per_tile // IDX_CHUNK, chunk_body, 0)
    plsc.subcore_barrier()

    def wb(k, _):
      base = s * per + k * STREAM
      pltpu.sync_copy(acc_sh.at[pl.ds(base, STREAM)], ones_v)
      pltpu.sync_copy(ones_v, deg_out.at[c, pl.ds(base, STREAM)])
      return 0
    lax.fori_loop(0, per // STREAM, wb, 0)

  return deg(dst2d)


def _dotT(a, w):
  return lax.dot_general(a, w, (((1,), (1,)), ((), ())),
                         preferred_element_type=jnp.float32)


def _tc_layer(sums, degs, x, Wl, b, Wr):
  d_out = Wl.shape[0]

  def body(sums_ref, degs_ref, x_ref, wl_ref, b_ref, wr_ref, o_ref):
    i = pl.program_id(0)
    agg = sums_ref[0] + sums_ref[1]
    deg = degs_ref[0] + degs_ref[1]
    mean = agg / jnp.maximum(deg, 1.0)
    h = _dotT(mean, wl_ref[...]) + _dotT(x_ref[...], wr_ref[...]) + b_ref[...]
    h = jnp.maximum(h, 0.0)
    rows = i * BLK + lax.broadcasted_iota(jnp.int32, (BLK, 1), 0)
    o_ref[...] = jnp.where(rows < N_NODES, h, 0.0)

  return pl.pallas_call(
      body,
      grid=(N_PAD // BLK,),
      in_specs=[
          pl.BlockSpec((N_SC, BLK, D), lambda i: (0, i, 0)),
          pl.BlockSpec((N_SC, BLK, D), lambda i: (0, i, 0)),
          pl.BlockSpec((BLK, D), lambda i: (i, 0)),
          pl.BlockSpec(Wl.shape, lambda i: (0, 0)),
          pl.BlockSpec((1, d_out), lambda i: (0, 0)),
          pl.BlockSpec(Wr.shape, lambda i: (0, 0)),
      ],
      out_specs=pl.BlockSpec((BLK, d_out), lambda i: (i, 0)),
      out_shape=jax.ShapeDtypeStruct((N_PAD, d_out), jnp.float32),
  )(sums, degs, x, Wl, b.reshape(1, -1), Wr)


def _tc_layer3_fc(sums, degs, x, Wl, b, Wr, Wfc_pad, bfc_pad):
  d_hid = Wl.shape[0]

  def body(sums_ref, degs_ref, x_ref, wl_ref, b_ref, wr_ref, wfc_ref,
           bfc_ref, o_ref):
    agg = sums_ref[0] + sums_ref[1]
    deg = degs_ref[0] + degs_ref[1]
    mean = agg / jnp.maximum(deg, 1.0)
    h = _dotT(mean, wl_ref[...]) + _dotT(x_ref[...], wr_ref[...]) + b_ref[...]
    h = jnp.maximum(h, 0.0)
    o_ref[...] = _dotT(h, wfc_ref[...]) + bfc_ref[...]

  return pl.pallas_call(
      body,
      grid=(N_PAD // BLK,),
      in_specs=[
          pl.BlockSpec((N_SC, BLK, D), lambda i: (0, i, 0)),
          pl.BlockSpec((N_SC, BLK, D), lambda i: (0, i, 0)),
          pl.BlockSpec((BLK, D), lambda i: (i, 0)),
          pl.BlockSpec(Wl.shape, lambda i: (0, 0)),
          pl.BlockSpec((1, d_hid), lambda i: (0, 0)),
          pl.BlockSpec(Wr.shape, lambda i: (0, 0)),
          pl.BlockSpec(Wfc_pad.shape, lambda i: (0, 0)),
          pl.BlockSpec((1, 128), lambda i: (0, 0)),
      ],
      out_specs=pl.BlockSpec((BLK, 128), lambda i: (i, 0)),
      out_shape=jax.ShapeDtypeStruct((N_PAD, 128), jnp.float32),
  )(sums, degs, x, Wl, b.reshape(1, -1), Wr, Wfc_pad, bfc_pad.reshape(1, -1))


def kernel(x, edge_index, W1l, b1, W1r, W2l, b2, W2r, W3l, b3, W3r, Wfc, bfc):
  src = edge_index[0].astype(jnp.int32)
  dst = edge_index[1].astype(jnp.int32)
  n_edges = src.shape[0]

  chunk = N_TILES * STREAM * IDX_CHUNK
  rows_per_tile = -(-n_edges // chunk) * IDX_CHUNK
  e_pad = rows_per_tile * N_TILES * STREAM
  n_fill = e_pad - n_edges
  fill = (N_NODES + jnp.arange(n_fill, dtype=jnp.int32) % (N_PAD - N_NODES))
  src2d = jnp.concatenate([src, fill]).reshape(-1, STREAM)
  dst2d = jnp.concatenate([dst, fill]).reshape(-1, STREAM)

  x_pad = jnp.zeros((N_PAD, D), jnp.float32).at[:N_NODES].set(x)
  Wfc_pad = jnp.zeros((128, Wfc.shape[1]), jnp.float32).at[:Wfc.shape[0]].set(Wfc)
  bfc_pad = jnp.zeros((128,), jnp.float32).at[:bfc.shape[0]].set(bfc)

  degs = _sc_degree(dst2d, rows_per_tile)
  s1 = _sc_aggregate(x_pad, src2d, dst2d, rows_per_tile)
  h1 = _tc_layer(s1, degs, x_pad, W1l, b1, W1r)
  s2 = _sc_aggregate(h1, src2d, dst2d, rows_per_tile)
  h2 = _tc_layer(s2, degs, h1, W2l, b2, W2r)
  s3 = _sc_aggregate(h2, src2d, dst2d, rows_per_tile)
  out = _tc_layer3_fc(s3, degs, h2, W3l, b3, W3r, Wfc_pad, bfc_pad)
  return out[:N_NODES, :Wfc.shape[0]]


# speedup vs baseline: 9.0760x; 1.2817x over previous
import functools

import jax
import jax.numpy as jnp
from jax import lax
from jax.experimental import pallas as pl
from jax.experimental.pallas import tpu as pltpu
from jax.experimental.pallas import tpu_sc as plsc

N_NODES = 10000
N_PAD = 10240
D = 128
STREAM = 128
N_SC = 2
N_SUB = 16
N_TILES = N_SC * N_SUB
IDX_CHUNK = 8
BLK = 256


def _zero_buf(buf, rows, cols):
  def zrow(i, _):
    def zcol(j, _):
      buf[i, pl.ds(j * 16, 16)] = jnp.zeros((16,), jnp.float32)
      return 0
    lax.fori_loop(0, cols // 16, zcol, 0)
    return 0
  lax.fori_loop(0, rows, zrow, 0)


def _sc_aggregate(table, src2d, dst2d, rows_per_tile):
  mesh = plsc.VectorSubcoreMesh(core_axis_name="c", subcore_axis_name="s")
  per = N_PAD // N_SUB

  @functools.partial(
      pl.kernel,
      out_type=jax.ShapeDtypeStruct((N_SC, N_PAD, D), jnp.float32),
      mesh=mesh,
      scratch_types=[
          pltpu.VMEM((IDX_CHUNK, STREAM), jnp.int32),
          pltpu.VMEM((IDX_CHUNK, STREAM), jnp.int32),
          pltpu.VMEM((2, STREAM, D), jnp.float32),
          pltpu.VMEM_SHARED((N_PAD, D), jnp.float32),
          pltpu.SemaphoreType.DMA,
          pltpu.SemaphoreType.DMA,
          pltpu.SemaphoreType.DMA,
          pltpu.SemaphoreType.DMA,
      ],
  )
  def agg(table_hbm, src_hbm, dst_hbm, sum_out, src_v, dst_v, rows_v, acc_sh,
          gsem0, gsem1, ssem0, ssem1):
    c = lax.axis_index("c")
    s = lax.axis_index("s")
    wid = c * N_SUB + s
    gsem = (gsem0, gsem1)
    ssem = (ssem0, ssem1)

    _zero_buf(rows_v.at[0], STREAM, D)
    def zshared(k, _):
      pltpu.sync_copy(rows_v.at[0],
                      acc_sh.at[pl.ds(s * per + k * STREAM, STREAM)])
      return 0
    lax.fori_loop(0, per // STREAM, zshared, 0)
    plsc.subcore_barrier()

    row0 = wid * rows_per_tile
    def chunk_body(k, _):
      base = row0 + k * IDX_CHUNK
      pltpu.sync_copy(src_hbm.at[pl.ds(base, IDX_CHUNK)], src_v)
      pltpu.sync_copy(dst_hbm.at[pl.ds(base, IDX_CHUNK)], dst_v)
      gd = [None, None]
      sd = [None, None]
      gd[0] = pltpu.async_copy(table_hbm.at[src_v.at[0]], rows_v.at[0],
                               gsem[0])
      for g in range(IDX_CHUNK):
        b = g % 2
        nb = 1 - b
        if g + 1 < IDX_CHUNK:
          if sd[nb] is not None:
            sd[nb].wait()
          gd[nb] = pltpu.async_copy(table_hbm.at[src_v.at[g + 1]],
                                    rows_v.at[nb], gsem[nb])
        gd[b].wait()
        sd[b] = pltpu.async_copy(rows_v.at[b], acc_sh.at[dst_v.at[g]],
                                 ssem[b], add=True)
      sd[0].wait()
      sd[1].wait()
      return 0
    lax.fori_loop(0, rows_per_tile // IDX_CHUNK, chunk_body, 0)
    plsc.subcore_barrier()

    def wb(k, _):
      base = s * per + k * STREAM
      pltpu.sync_copy(acc_sh.at[pl.ds(base, STREAM)], rows_v.at[0])
      pltpu.sync_copy(rows_v.at[0], sum_out.at[c, pl.ds(base, STREAM)])
      return 0
    lax.fori_loop(0, per // STREAM, wb, 0)

  return agg(table, src2d, dst2d)


def _sc_degree(dst2d, rows_per_tile):
  mesh = plsc.VectorSubcoreMesh(core_axis_name="c", subcore_axis_name="s")
  per = N_PAD // N_SUB

  @functools.partial(
      pl.kernel,
      out_type=jax.ShapeDtypeStruct((N_SC, N_PAD, D), jnp.float32),
      mesh=mesh,
      scratch_types=[
          pltpu.VMEM((IDX_CHUNK, STREAM), jnp.int32),
          pltpu.VMEM((STREAM, D), jnp.float32),
          pltpu.VMEM_SHARED((N_PAD, D), jnp.float32),
          pltpu.SemaphoreType.DMA,
      ],
  )
  def deg(dst_hbm, deg_out, dst_v, ones_v, acc_sh, ssem):
    c = lax.axis_index("c")
    s = lax.axis_index("s")
    wid = c * N_SUB + s

    _zero_buf(ones_v, STREAM, D)
    def zshared(k, _):
      pltpu.sync_copy(ones_v, acc_sh.at[pl.ds(s * per + k * STREAM, STREAM)])
      return 0
    lax.fori_loop(0, per // STREAM, zshared, 0)

    def orow(i, _):
      def ocol(j, _):
        ones_v[i, pl.ds(j * 16, 16)] = jnp.ones((16,), jnp.float32)
        return 0
      lax.fori_loop(0, D // 16, ocol, 0)
      return 0
    lax.fori_loop(0, STREAM, orow, 0)
    plsc.subcore_barrier()

    row0 = wid * rows_per_tile
    def chunk_body(k, _):
      base = row0 + k * IDX_CHUNK
      pltpu.sync_copy(dst_hbm.at[pl.ds(base, IDX_CHUNK)], dst_v)
      sds = [pltpu.async_copy(ones_v, acc_sh.at[dst_v.at[g]], ssem, add=True)
             for g in range(IDX_CHUNK)]
      for d in sds:
        d.wait()
      return 0
    lax.fori_loop(0, rows_per_tile // IDX_CHUNK, chunk_body, 0)
    plsc.subcore_barrier()

    def wb(k, _):
      base = s * per + k * STREAM
      pltpu.sync_copy(acc_sh.at[pl.ds(base, STREAM)], ones_v)
      pltpu.sync_copy(ones_v, deg_out.at[c, pl.ds(base, STREAM)])
      return 0
    lax.fori_loop(0, per // STREAM, wb, 0)

  return deg(dst2d)


def _dotT(a, w):
  return lax.dot_general(a, w, (((1,), (1,)), ((), ())),
                         preferred_element_type=jnp.float32)


def _tc_layer(sums, degs, x, Wl, b, Wr):
  d_out = Wl.shape[0]

  def body(sums_ref, degs_ref, x_ref, wl_ref, b_ref, wr_ref, o_ref):
    i = pl.program_id(0)
    agg = sums_ref[0] + sums_ref[1]
    deg = degs_ref[0] + degs_ref[1]
    mean = agg / jnp.maximum(deg, 1.0)
    h = _dotT(mean, wl_ref[...]) + _dotT(x_ref[...], wr_ref[...]) + b_ref[...]
    h = jnp.maximum(h, 0.0)
    rows = i * BLK + lax.broadcasted_iota(jnp.int32, (BLK, 1), 0)
    o_ref[...] = jnp.where(rows < N_NODES, h, 0.0)

  return pl.pallas_call(
      body,
      grid=(N_PAD // BLK,),
      in_specs=[
          pl.BlockSpec((N_SC, BLK, D), lambda i: (0, i, 0)),
          pl.BlockSpec((N_SC, BLK, D), lambda i: (0, i, 0)),
          pl.BlockSpec((BLK, D), lambda i: (i, 0)),
          pl.BlockSpec(Wl.shape, lambda i: (0, 0)),
          pl.BlockSpec((1, d_out), lambda i: (0, 0)),
          pl.BlockSpec(Wr.shape, lambda i: (0, 0)),
      ],
      out_specs=pl.BlockSpec((BLK, d_out), lambda i: (i, 0)),
      out_shape=jax.ShapeDtypeStruct((N_PAD, d_out), jnp.float32),
  )(sums, degs, x, Wl, b.reshape(1, -1), Wr)


def _tc_layer3_fc(sums, degs, x, Wl, b, Wr, Wfc_pad, bfc_pad):
  d_hid = Wl.shape[0]

  def body(sums_ref, degs_ref, x_ref, wl_ref, b_ref, wr_ref, wfc_ref,
           bfc_ref, o_ref):
    agg = sums_ref[0] + sums_ref[1]
    deg = degs_ref[0] + degs_ref[1]
    mean = agg / jnp.maximum(deg, 1.0)
    h = _dotT(mean, wl_ref[...]) + _dotT(x_ref[...], wr_ref[...]) + b_ref[...]
    h = jnp.maximum(h, 0.0)
    o_ref[...] = _dotT(h, wfc_ref[...]) + bfc_ref[...]

  return pl.pallas_call(
      body,
      grid=(N_PAD // BLK,),
      in_specs=[
          pl.BlockSpec((N_SC, BLK, D), lambda i: (0, i, 0)),
          pl.BlockSpec((N_SC, BLK, D), lambda i: (0, i, 0)),
          pl.BlockSpec((BLK, D), lambda i: (i, 0)),
          pl.BlockSpec(Wl.shape, lambda i: (0, 0)),
          pl.BlockSpec((1, d_hid), lambda i: (0, 0)),
          pl.BlockSpec(Wr.shape, lambda i: (0, 0)),
          pl.BlockSpec(Wfc_pad.shape, lambda i: (0, 0)),
          pl.BlockSpec((1, 128), lambda i: (0, 0)),
      ],
      out_specs=pl.BlockSpec((BLK, 128), lambda i: (i, 0)),
      out_shape=jax.ShapeDtypeStruct((N_PAD, 128), jnp.float32),
  )(sums, degs, x, Wl, b.reshape(1, -1), Wr, Wfc_pad, bfc_pad.reshape(1, -1))


def kernel(x, edge_index, W1l, b1, W1r, W2l, b2, W2r, W3l, b3, W3r, Wfc, bfc):
  src = edge_index[0].astype(jnp.int32)
  dst = edge_index[1].astype(jnp.int32)
  n_edges = src.shape[0]

  chunk = N_TILES * STREAM * IDX_CHUNK
  rows_per_tile = -(-n_edges // chunk) * IDX_CHUNK
  e_pad = rows_per_tile * N_TILES * STREAM
  n_fill = e_pad - n_edges
  fill = (N_NODES + jnp.arange(n_fill, dtype=jnp.int32) % (N_PAD - N_NODES))
  src2d = jnp.concatenate([src, fill]).reshape(-1, STREAM)
  dst2d = jnp.concatenate([dst, fill]).reshape(-1, STREAM)

  x_pad = jnp.zeros((N_PAD, D), jnp.float32).at[:N_NODES].set(x)
  Wfc_pad = jnp.zeros((128, Wfc.shape[1]), jnp.float32).at[:Wfc.shape[0]].set(Wfc)
  bfc_pad = jnp.zeros((128,), jnp.float32).at[:bfc.shape[0]].set(bfc)

  degs = _sc_degree(dst2d, rows_per_tile)
  s1 = _sc_aggregate(x_pad, src2d, dst2d, rows_per_tile)
  h1 = _tc_layer(s1, degs, x_pad, W1l, b1, W1r)
  s2 = _sc_aggregate(h1, src2d, dst2d, rows_per_tile)
  h2 = _tc_layer(s2, degs, h1, W2l, b2, W2r)
  s3 = _sc_aggregate(h2, src2d, dst2d, rows_per_tile)
  out = _tc_layer3_fc(s3, degs, h2, W3l, b3, W3r, Wfc_pad, bfc_pad)
  return out[:N_NODES, :Wfc.shape[0]]


# speedup vs baseline: 9.6653x; 1.0649x over previous
import functools

import jax
import jax.numpy as jnp
from jax import lax
from jax.experimental import pallas as pl
from jax.experimental.pallas import tpu as pltpu
from jax.experimental.pallas import tpu_sc as plsc

N_NODES = 10000
N_PAD = 10240
D = 128
STREAM = 128
N_SC = 2
N_SUB = 16
N_TILES = N_SC * N_SUB
IDX_CHUNK = 16
BLK = 256


def _zero_buf(buf, rows, cols):
  def zrow(i, _):
    def zcol(j, _):
      buf[i, pl.ds(j * 16, 16)] = jnp.zeros((16,), jnp.float32)
      return 0
    lax.fori_loop(0, cols // 16, zcol, 0)
    return 0
  lax.fori_loop(0, rows, zrow, 0)


def _sc_aggregate(table, src2d, dst2d, rows_per_tile):
  mesh = plsc.VectorSubcoreMesh(core_axis_name="c", subcore_axis_name="s")
  per = N_PAD // N_SUB

  @functools.partial(
      pl.kernel,
      out_type=jax.ShapeDtypeStruct((N_SC, N_PAD, D), jnp.float32),
      mesh=mesh,
      scratch_types=[
          pltpu.VMEM((IDX_CHUNK, STREAM), jnp.int32),
          pltpu.VMEM((IDX_CHUNK, STREAM), jnp.int32),
          pltpu.VMEM((2, STREAM, D), jnp.float32),
          pltpu.VMEM_SHARED((N_PAD, D), jnp.float32),
          pltpu.SemaphoreType.DMA,
          pltpu.SemaphoreType.DMA,
          pltpu.SemaphoreType.DMA,
          pltpu.SemaphoreType.DMA,
      ],
  )
  def agg(table_hbm, src_hbm, dst_hbm, sum_out, src_v, dst_v, rows_v, acc_sh,
          gsem0, gsem1, ssem0, ssem1):
    c = lax.axis_index("c")
    s = lax.axis_index("s")
    wid = c * N_SUB + s
    gsem = (gsem0, gsem1)
    ssem = (ssem0, ssem1)

    _zero_buf(rows_v.at[0], STREAM, D)
    def zshared(k, _):
      pltpu.sync_copy(rows_v.at[0],
                      acc_sh.at[pl.ds(s * per + k * STREAM, STREAM)])
      return 0
    lax.fori_loop(0, per // STREAM, zshared, 0)
    plsc.subcore_barrier()

    row0 = wid * rows_per_tile
    def chunk_body(k, _):
      base = row0 + k * IDX_CHUNK
      pltpu.sync_copy(src_hbm.at[pl.ds(base, IDX_CHUNK)], src_v)
      pltpu.sync_copy(dst_hbm.at[pl.ds(base, IDX_CHUNK)], dst_v)
      gd = [None, None]
      sd = [None, None]
      gd[0] = pltpu.async_copy(table_hbm.at[src_v.at[0]], rows_v.at[0],
                               gsem[0])
      for g in range(IDX_CHUNK):
        b = g % 2
        nb = 1 - b
        if g + 1 < IDX_CHUNK:
          if sd[nb] is not None:
            sd[nb].wait()
          gd[nb] = pltpu.async_copy(table_hbm.at[src_v.at[g + 1]],
                                    rows_v.at[nb], gsem[nb])
        gd[b].wait()
        sd[b] = pltpu.async_copy(rows_v.at[b], acc_sh.at[dst_v.at[g]],
                                 ssem[b], add=True)
      sd[0].wait()
      sd[1].wait()
      return 0
    lax.fori_loop(0, rows_per_tile // IDX_CHUNK, chunk_body, 0)
    plsc.subcore_barrier()

    def wb(k, _):
      base = s * per + k * STREAM
      pltpu.sync_copy(acc_sh.at[pl.ds(base, STREAM)], rows_v.at[0])
      pltpu.sync_copy(rows_v.at[0], sum_out.at[c, pl.ds(base, STREAM)])
      return 0
    lax.fori_loop(0, per // STREAM, wb, 0)

  return agg(table, src2d, dst2d)


def _sc_degree(dst2d, rows_per_tile):
  mesh = plsc.VectorSubcoreMesh(core_axis_name="c", subcore_axis_name="s")
  per = N_PAD // N_SUB

  @functools.partial(
      pl.kernel,
      out_type=jax.ShapeDtypeStruct((N_SC, N_PAD, D), jnp.float32),
      mesh=mesh,
      scratch_types=[
          pltpu.VMEM((IDX_CHUNK, STREAM), jnp.int32),
          pltpu.VMEM((STREAM, D), jnp.float32),
          pltpu.VMEM_SHARED((N_PAD, D), jnp.float32),
          pltpu.SemaphoreType.DMA,
      ],
  )
  def deg(dst_hbm, deg_out, dst_v, ones_v, acc_sh, ssem):
    c = lax.axis_index("c")
    s = lax.axis_index("s")
    wid = c * N_SUB + s

    _zero_buf(ones_v, STREAM, D)
    def zshared(k, _):
      pltpu.sync_copy(ones_v, acc_sh.at[pl.ds(s * per + k * STREAM, STREAM)])
      return 0
    lax.fori_loop(0, per // STREAM, zshared, 0)

    def orow(i, _):
      def ocol(j, _):
        ones_v[i, pl.ds(j * 16, 16)] = jnp.ones((16,), jnp.float32)
        return 0
      lax.fori_loop(0, D // 16, ocol, 0)
      return 0
    lax.fori_loop(0, STREAM, orow, 0)
    plsc.subcore_barrier()

    row0 = wid * rows_per_tile
    def chunk_body(k, _):
      base = row0 + k * IDX_CHUNK
      pltpu.sync_copy(dst_hbm.at[pl.ds(base, IDX_CHUNK)], dst_v)
      sds = [pltpu.async_copy(ones_v, acc_sh.at[dst_v.at[g]], ssem, add=True)
             for g in range(IDX_CHUNK)]
      for d in sds:
        d.wait()
      return 0
    lax.fori_loop(0, rows_per_tile // IDX_CHUNK, chunk_body, 0)
    plsc.subcore_barrier()

    def wb(k, _):
      base = s * per + k * STREAM
      pltpu.sync_copy(acc_sh.at[pl.ds(base, STREAM)], ones_v)
      pltpu.sync_copy(ones_v, deg_out.at[c, pl.ds(base, STREAM)])
      return 0
    lax.fori_loop(0, per // STREAM, wb, 0)

  return deg(dst2d)


def _dotT(a, w):
  return lax.dot_general(a, w, (((1,), (1,)), ((), ())),
                         preferred_element_type=jnp.float32)


def _tc_layer(sums, degs, x, Wl, b, Wr):
  d_out = Wl.shape[0]

  def body(sums_ref, degs_ref, x_ref, wl_ref, b_ref, wr_ref, o_ref):
    i = pl.program_id(0)
    agg = sums_ref[0] + sums_ref[1]
    deg = degs_ref[0] + degs_ref[1]
    mean = agg / jnp.maximum(deg, 1.0)
    h = _dotT(mean, wl_ref[...]) + _dotT(x_ref[...], wr_ref[...]) + b_ref[...]
    h = jnp.maximum(h, 0.0)
    rows = i * BLK + lax.broadcasted_iota(jnp.int32, (BLK, 1), 0)
    o_ref[...] = jnp.where(rows < N_NODES, h, 0.0)

  return pl.pallas_call(
      body,
      grid=(N_PAD // BLK,),
      in_specs=[
          pl.BlockSpec((N_SC, BLK, D), lambda i: (0, i, 0)),
          pl.BlockSpec((N_SC, BLK, D), lambda i: (0, i, 0)),
          pl.BlockSpec((BLK, D), lambda i: (i, 0)),
          pl.BlockSpec(Wl.shape, lambda i: (0, 0)),
          pl.BlockSpec((1, d_out), lambda i: (0, 0)),
          pl.BlockSpec(Wr.shape, lambda i: (0, 0)),
      ],
      out_specs=pl.BlockSpec((BLK, d_out), lambda i: (i, 0)),
      out_shape=jax.ShapeDtypeStruct((N_PAD, d_out), jnp.float32),
  )(sums, degs, x, Wl, b.reshape(1, -1), Wr)


def _tc_layer3_fc(sums, degs, x, Wl, b, Wr, Wfc_pad, bfc_pad):
  d_hid = Wl.shape[0]

  def body(sums_ref, degs_ref, x_ref, wl_ref, b_ref, wr_ref, wfc_ref,
           bfc_ref, o_ref):
    agg = sums_ref[0] + sums_ref[1]
    deg = degs_ref[0] + degs_ref[1]
    mean = agg / jnp.maximum(deg, 1.0)
    h = _dotT(mean, wl_ref[...]) + _dotT(x_ref[...], wr_ref[...]) + b_ref[...]
    h = jnp.maximum(h, 0.0)
    o_ref[...] = _dotT(h, wfc_ref[...]) + bfc_ref[...]

  return pl.pallas_call(
      body,
      grid=(N_PAD // BLK,),
      in_specs=[
          pl.BlockSpec((N_SC, BLK, D), lambda i: (0, i, 0)),
          pl.BlockSpec((N_SC, BLK, D), lambda i: (0, i, 0)),
          pl.BlockSpec((BLK, D), lambda i: (i, 0)),
          pl.BlockSpec(Wl.shape, lambda i: (0, 0)),
          pl.BlockSpec((1, d_hid), lambda i: (0, 0)),
          pl.BlockSpec(Wr.shape, lambda i: (0, 0)),
          pl.BlockSpec(Wfc_pad.shape, lambda i: (0, 0)),
          pl.BlockSpec((1, 128), lambda i: (0, 0)),
      ],
      out_specs=pl.BlockSpec((BLK, 128), lambda i: (i, 0)),
      out_shape=jax.ShapeDtypeStruct((N_PAD, 128), jnp.float32),
  )(sums, degs, x, Wl, b.reshape(1, -1), Wr, Wfc_pad, bfc_pad.reshape(1, -1))


def kernel(x, edge_index, W1l, b1, W1r, W2l, b2, W2r, W3l, b3, W3r, Wfc, bfc):
  src = edge_index[0].astype(jnp.int32)
  dst = edge_index[1].astype(jnp.int32)
  n_edges = src.shape[0]

  chunk = N_TILES * STREAM * IDX_CHUNK
  rows_per_tile = -(-n_edges // chunk) * IDX_CHUNK
  e_pad = rows_per_tile * N_TILES * STREAM
  n_fill = e_pad - n_edges
  fill = (N_NODES + jnp.arange(n_fill, dtype=jnp.int32) % (N_PAD - N_NODES))
  src2d = jnp.concatenate([src, fill]).reshape(-1, STREAM)
  dst2d = jnp.concatenate([dst, fill]).reshape(-1, STREAM)

  x_pad = jnp.zeros((N_PAD, D), jnp.float32).at[:N_NODES].set(x)
  Wfc_pad = jnp.zeros((128, Wfc.shape[1]), jnp.float32).at[:Wfc.shape[0]].set(Wfc)
  bfc_pad = jnp.zeros((128,), jnp.float32).at[:bfc.shape[0]].set(bfc)

  degs = _sc_degree(dst2d, rows_per_tile)
  s1 = _sc_aggregate(x_pad, src2d, dst2d, rows_per_tile)
  h1 = _tc_layer(s1, degs, x_pad, W1l, b1, W1r)
  s2 = _sc_aggregate(h1, src2d, dst2d, rows_per_tile)
  h2 = _tc_layer(s2, degs, h1, W2l, b2, W2r)
  s3 = _sc_aggregate(h2, src2d, dst2d, rows_per_tile)
  out = _tc_layer3_fc(s3, degs, h2, W3l, b3, W3r, Wfc_pad, bfc_pad)
  return out[:N_NODES, :Wfc.shape[0]]


# speedup vs baseline: 9.9488x; 1.0293x over previous
import functools

import jax
import jax.numpy as jnp
from jax import lax
from jax.experimental import pallas as pl
from jax.experimental.pallas import tpu as pltpu
from jax.experimental.pallas import tpu_sc as plsc

N_NODES = 10000
N_PAD = 10240
D = 128
STREAM = 128
N_SC = 2
N_SUB = 16
N_TILES = N_SC * N_SUB
IDX_CHUNK = 8
PAIR = 2 * IDX_CHUNK
BLK = 256


def _zero_buf(buf, rows, cols):
  def zrow(i, _):
    def zcol(j, _):
      buf[i, pl.ds(j * 16, 16)] = jnp.zeros((16,), jnp.float32)
      return 0
    lax.fori_loop(0, cols // 16, zcol, 0)
    return 0
  lax.fori_loop(0, rows, zrow, 0)


def _sc_aggregate(table, src2d, dst2d, rows_per_tile):
  mesh = plsc.VectorSubcoreMesh(core_axis_name="c", subcore_axis_name="s")
  per = N_PAD // N_SUB

  @functools.partial(
      pl.kernel,
      out_type=jax.ShapeDtypeStruct((N_SC, N_PAD, D), jnp.float32),
      mesh=mesh,
      scratch_types=[
          pltpu.VMEM((2, IDX_CHUNK, STREAM), jnp.int32),
          pltpu.VMEM((2, IDX_CHUNK, STREAM), jnp.int32),
          pltpu.VMEM((2, STREAM, D), jnp.float32),
          pltpu.VMEM_SHARED((N_PAD, D), jnp.float32),
          pltpu.SemaphoreType.DMA,
          pltpu.SemaphoreType.DMA,
          pltpu.SemaphoreType.DMA,
          pltpu.SemaphoreType.DMA,
          pltpu.SemaphoreType.DMA,
          pltpu.SemaphoreType.DMA,
      ],
  )
  def agg(table_hbm, src_hbm, dst_hbm, sum_out, src_v, dst_v, rows_v, acc_sh,
          gsem0, gsem1, ssem0, ssem1, isem0, isem1):
    c = lax.axis_index("c")
    s = lax.axis_index("s")
    wid = c * N_SUB + s
    gsem = (gsem0, gsem1)
    ssem = (ssem0, ssem1)
    isem = (isem0, isem1)

    _zero_buf(rows_v.at[0], STREAM, D)
    def zshared(k, _):
      pltpu.sync_copy(rows_v.at[0],
                      acc_sh.at[pl.ds(s * per + k * STREAM, STREAM)])
      return 0
    lax.fori_loop(0, per // STREAM, zshared, 0)
    plsc.subcore_barrier()

    row0 = wid * rows_per_tile
    n_chunks = rows_per_tile // IDX_CHUNK

    def idx_fetch(slot, base):
      pltpu.async_copy(src_hbm.at[pl.ds(base, IDX_CHUNK)], src_v.at[slot],
                       isem[slot])
      pltpu.async_copy(dst_hbm.at[pl.ds(base, IDX_CHUNK)], dst_v.at[slot],
                       isem[slot])

    def idx_drain(slot):
      pltpu.make_async_copy(src_hbm.at[pl.ds(row0, IDX_CHUNK)],
                            src_v.at[slot], isem[slot]).wait()
      pltpu.make_async_copy(dst_hbm.at[pl.ds(row0, IDX_CHUNK)],
                            dst_v.at[slot], isem[slot]).wait()

    idx_fetch(0, row0)
    idx_fetch(1, row0 + IDX_CHUNK)

    def pair_body(p, _):
      pre0 = row0 + jnp.minimum(2 * p + 2, n_chunks - 1) * IDX_CHUNK
      pre1 = row0 + jnp.minimum(2 * p + 3, n_chunks - 1) * IDX_CHUNK
      idx_drain(0)
      total = PAIR
      gd = [None] * (total + 1)
      sd = [None] * total
      gd[0] = pltpu.async_copy(table_hbm.at[src_v.at[0].at[0]], rows_v.at[0],
                               gsem[0])
      for t in range(total):
        b = t % 2
        nb = 1 - b
        if t + 1 < total:
          if t + 1 == IDX_CHUNK:
            idx_drain(1)
          if t >= 1:
            sd[t - 1].wait()
          sl, g = (t + 1) // IDX_CHUNK, (t + 1) % IDX_CHUNK
          gd[t + 1] = pltpu.async_copy(table_hbm.at[src_v.at[sl].at[g]],
                                       rows_v.at[nb], gsem[nb])
        gd[t].wait()
        sl, g = t // IDX_CHUNK, t % IDX_CHUNK
        sd[t] = pltpu.async_copy(rows_v.at[b], acc_sh.at[dst_v.at[sl].at[g]],
                                 ssem[b], add=True)
        if t == IDX_CHUNK:
          idx_fetch(0, pre0)
      sd[total - 2].wait()
      sd[total - 1].wait()
      idx_fetch(1, pre1)
      return 0
    lax.fori_loop(0, n_chunks // 2, pair_body, 0)
    idx_drain(0)
    idx_drain(1)
    plsc.subcore_barrier()

    def wb(k, _):
      base = s * per + k * STREAM
      pltpu.sync_copy(acc_sh.at[pl.ds(base, STREAM)], rows_v.at[0])
      pltpu.sync_copy(rows_v.at[0], sum_out.at[c, pl.ds(base, STREAM)])
      return 0
    lax.fori_loop(0, per // STREAM, wb, 0)

  return agg(table, src2d, dst2d)


def _sc_degree(dst2d, rows_per_tile):
  mesh = plsc.VectorSubcoreMesh(core_axis_name="c", subcore_axis_name="s")
  per = N_PAD // N_SUB

  @functools.partial(
      pl.kernel,
      out_type=jax.ShapeDtypeStruct((N_SC, N_PAD, D), jnp.float32),
      mesh=mesh,
      scratch_types=[
          pltpu.VMEM((IDX_CHUNK, STREAM), jnp.int32),
          pltpu.VMEM((STREAM, D), jnp.float32),
          pltpu.VMEM_SHARED((N_PAD, D), jnp.float32),
          pltpu.SemaphoreType.DMA,
      ],
  )
  def deg(dst_hbm, deg_out, dst_v, ones_v, acc_sh, ssem):
    c = lax.axis_index("c")
    s = lax.axis_index("s")
    wid = c * N_SUB + s

    _zero_buf(ones_v, STREAM, D)
    def zshared(k, _):
      pltpu.sync_copy(ones_v, acc_sh.at[pl.ds(s * per + k * STREAM, STREAM)])
      return 0
    lax.fori_loop(0, per // STREAM, zshared, 0)

    def orow(i, _):
      def ocol(j, _):
        ones_v[i, pl.ds(j * 16, 16)] = jnp.ones((16,), jnp.float32)
        return 0
      lax.fori_loop(0, D // 16, ocol, 0)
      return 0
    lax.fori_loop(0, STREAM, orow, 0)
    plsc.subcore_barrier()

    row0 = wid * rows_per_tile
    def chunk_body(k, _):
      base = row0 + k * IDX_CHUNK
      pltpu.sync_copy(dst_hbm.at[pl.ds(base, IDX_CHUNK)], dst_v)
      sds = [pltpu.async_copy(ones_v, acc_sh.at[dst_v.at[g]], ssem, add=True)
             for g in range(IDX_CHUNK)]
      for d in sds:
        d.wait()
      return 0
    lax.fori_loop(0, rows_per_tile // IDX_CHUNK, chunk_body, 0)
    plsc.subcore_barrier()

    def wb(k, _):
      base = s * per + k * STREAM
      pltpu.sync_copy(acc_sh.at[pl.ds(base, STREAM)], ones_v)
      pltpu.sync_copy(ones_v, deg_out.at[c, pl.ds(base, STREAM)])
      return 0
    lax.fori_loop(0, per // STREAM, wb, 0)

  return deg(dst2d)


def _dotT(a, w):
  return lax.dot_general(a, w, (((1,), (1,)), ((), ())),
                         preferred_element_type=jnp.float32)


def _tc_layer(sums, degs, x, Wl, b, Wr):
  d_out = Wl.shape[0]

  def body(sums_ref, degs_ref, x_ref, wl_ref, b_ref, wr_ref, o_ref):
    i = pl.program_id(0)
    agg = sums_ref[0] + sums_ref[1]
    deg = degs_ref[0] + degs_ref[1]
    mean = agg / jnp.maximum(deg, 1.0)
    h = _dotT(mean, wl_ref[...]) + _dotT(x_ref[...], wr_ref[...]) + b_ref[...]
    h = jnp.maximum(h, 0.0)
    rows = i * BLK + lax.broadcasted_iota(jnp.int32, (BLK, 1), 0)
    o_ref[...] = jnp.where(rows < N_NODES, h, 0.0)

  return pl.pallas_call(
      body,
      grid=(N_PAD // BLK,),
      in_specs=[
          pl.BlockSpec((N_SC, BLK, D), lambda i: (0, i, 0)),
          pl.BlockSpec((N_SC, BLK, D), lambda i: (0, i, 0)),
          pl.BlockSpec((BLK, D), lambda i: (i, 0)),
          pl.BlockSpec(Wl.shape, lambda i: (0, 0)),
          pl.BlockSpec((1, d_out), lambda i: (0, 0)),
          pl.BlockSpec(Wr.shape, lambda i: (0, 0)),
      ],
      out_specs=pl.BlockSpec((BLK, d_out), lambda i: (i, 0)),
      out_shape=jax.ShapeDtypeStruct((N_PAD, d_out), jnp.float32),
  )(sums, degs, x, Wl, b.reshape(1, -1), Wr)


def _tc_layer3_fc(sums, degs, x, Wl, b, Wr, Wfc_pad, bfc_pad):
  d_hid = Wl.shape[0]

  def body(sums_ref, degs_ref, x_ref, wl_ref, b_ref, wr_ref, wfc_ref,
           bfc_ref, o_ref):
    agg = sums_ref[0] + sums_ref[1]
    deg = degs_ref[0] + degs_ref[1]
    mean = agg / jnp.maximum(deg, 1.0)
    h = _dotT(mean, wl_ref[...]) + _dotT(x_ref[...], wr_ref[...]) + b_ref[...]
    h = jnp.maximum(h, 0.0)
    o_ref[...] = _dotT(h, wfc_ref[...]) + bfc_ref[...]

  return pl.pallas_call(
      body,
      grid=(N_PAD // BLK,),
      in_specs=[
          pl.BlockSpec((N_SC, BLK, D), lambda i: (0, i, 0)),
          pl.BlockSpec((N_SC, BLK, D), lambda i: (0, i, 0)),
          pl.BlockSpec((BLK, D), lambda i: (i, 0)),
          pl.BlockSpec(Wl.shape, lambda i: (0, 0)),
          pl.BlockSpec((1, d_hid), lambda i: (0, 0)),
          pl.BlockSpec(Wr.shape, lambda i: (0, 0)),
          pl.BlockSpec(Wfc_pad.shape, lambda i: (0, 0)),
          pl.BlockSpec((1, 128), lambda i: (0, 0)),
      ],
      out_specs=pl.BlockSpec((BLK, 128), lambda i: (i, 0)),
      out_shape=jax.ShapeDtypeStruct((N_PAD, 128), jnp.float32),
  )(sums, degs, x, Wl, b.reshape(1, -1), Wr, Wfc_pad, bfc_pad.reshape(1, -1))


def kernel(x, edge_index, W1l, b1, W1r, W2l, b2, W2r, W3l, b3, W3r, Wfc, bfc):
  src = edge_index[0].astype(jnp.int32)
  dst = edge_index[1].astype(jnp.int32)
  n_edges = src.shape[0]

  chunk = N_TILES * STREAM * PAIR
  rows_per_tile = -(-n_edges // chunk) * PAIR
  e_pad = rows_per_tile * N_TILES * STREAM
  n_fill = e_pad - n_edges
  fill = (N_NODES + jnp.arange(n_fill, dtype=jnp.int32) % (N_PAD - N_NODES))
  src2d = jnp.concatenate([src, fill]).reshape(-1, STREAM)
  dst2d = jnp.concatenate([dst, fill]).reshape(-1, STREAM)

  x_pad = jnp.zeros((N_PAD, D), jnp.float32).at[:N_NODES].set(x)
  Wfc_pad = jnp.zeros((128, Wfc.shape[1]), jnp.float32).at[:Wfc.shape[0]].set(Wfc)
  bfc_pad = jnp.zeros((128,), jnp.float32).at[:bfc.shape[0]].set(bfc)

  degs = _sc_degree(dst2d, rows_per_tile)
  s1 = _sc_aggregate(x_pad, src2d, dst2d, rows_per_tile)
  h1 = _tc_layer(s1, degs, x_pad, W1l, b1, W1r)
  s2 = _sc_aggregate(h1, src2d, dst2d, rows_per_tile)
  h2 = _tc_layer(s2, degs, h1, W2l, b2, W2r)
  s3 = _sc_aggregate(h2, src2d, dst2d, rows_per_tile)
  out = _tc_layer3_fc(s3, degs, h2, W3l, b3, W3r, Wfc_pad, bfc_pad)
  return out[:N_NODES, :Wfc.shape[0]]


# speedup vs baseline: 10.0760x; 1.0128x over previous
import functools

import jax
import jax.numpy as jnp
from jax import lax
from jax.experimental import pallas as pl
from jax.experimental.pallas import tpu as pltpu
from jax.experimental.pallas import tpu_sc as plsc

N_NODES = 10000
N_PAD = 10240
D = 128
STREAM = 128
N_SC = 2
N_SUB = 16
N_TILES = N_SC * N_SUB
IDX_CHUNK = 8
PAIR = 2 * IDX_CHUNK
BLK = 256


def _zero_buf(buf, rows, cols):
  def zrow(i, _):
    def zcol(j, _):
      buf[i, pl.ds(j * 16, 16)] = jnp.zeros((16,), jnp.float32)
      return 0
    lax.fori_loop(0, cols // 16, zcol, 0)
    return 0
  lax.fori_loop(0, rows, zrow, 0)


def _ones_buf(buf, rows, cols):
  def orow(i, _):
    def ocol(j, _):
      buf[i, pl.ds(j * 16, 16)] = jnp.ones((16,), jnp.float32)
      return 0
    lax.fori_loop(0, cols // 16, ocol, 0)
    return 0
  lax.fori_loop(0, rows, orow, 0)


def _sc_aggregate(table, src2d, dst2d, rows_per_tile, with_deg=False):
  mesh = plsc.VectorSubcoreMesh(core_axis_name="c", subcore_axis_name="s")
  per = N_PAD // N_SUB
  out_type = jax.ShapeDtypeStruct((N_SC, N_PAD, D), jnp.float32)

  @functools.partial(
      pl.kernel,
      out_type=[out_type, out_type] if with_deg else out_type,
      mesh=mesh,
      scratch_types=[
          pltpu.VMEM((2, IDX_CHUNK, STREAM), jnp.int32),
          pltpu.VMEM((2, IDX_CHUNK, STREAM), jnp.int32),
          pltpu.VMEM((2, STREAM, D), jnp.float32),
          pltpu.VMEM_SHARED((N_PAD, D), jnp.float32),
          pltpu.SemaphoreType.DMA,
          pltpu.SemaphoreType.DMA,
          pltpu.SemaphoreType.DMA,
          pltpu.SemaphoreType.DMA,
          pltpu.SemaphoreType.DMA,
          pltpu.SemaphoreType.DMA,
      ],
  )
  def agg(table_hbm, src_hbm, dst_hbm, *out_and_scratch):
    if with_deg:
      (sum_out, deg_out, src_v, dst_v, rows_v, acc_sh,
       gsem0, gsem1, ssem0, ssem1, isem0, isem1) = out_and_scratch
    else:
      (sum_out, src_v, dst_v, rows_v, acc_sh,
       gsem0, gsem1, ssem0, ssem1, isem0, isem1) = out_and_scratch
    c = lax.axis_index("c")
    s = lax.axis_index("s")
    wid = c * N_SUB + s
    gsem = (gsem0, gsem1)
    ssem = (ssem0, ssem1)
    isem = (isem0, isem1)
    row0 = wid * rows_per_tile

    _zero_buf(rows_v.at[0], STREAM, D)
    def zshared(k, _):
      pltpu.sync_copy(rows_v.at[0],
                      acc_sh.at[pl.ds(s * per + k * STREAM, STREAM)])
      return 0

    if with_deg:
      _ones_buf(rows_v.at[1], STREAM, D)
      lax.fori_loop(0, per // STREAM, zshared, 0)
      plsc.subcore_barrier()
      def deg_chunk(k, _):
        base = row0 + k * IDX_CHUNK
        pltpu.sync_copy(dst_hbm.at[pl.ds(base, IDX_CHUNK)], dst_v.at[0])
        sds = [pltpu.async_copy(rows_v.at[1], acc_sh.at[dst_v.at[0].at[g]],
                                ssem0, add=True)
               for g in range(IDX_CHUNK)]
        for d in sds:
          d.wait()
        return 0
      lax.fori_loop(0, rows_per_tile // IDX_CHUNK, deg_chunk, 0)
      plsc.subcore_barrier()
      def deg_wb(k, _):
        base = s * per + k * STREAM
        pltpu.sync_copy(acc_sh.at[pl.ds(base, STREAM)], rows_v.at[1])
        pltpu.sync_copy(rows_v.at[1], deg_out.at[c, pl.ds(base, STREAM)])
        return 0
      lax.fori_loop(0, per // STREAM, deg_wb, 0)

    lax.fori_loop(0, per // STREAM, zshared, 0)
    plsc.subcore_barrier()

    row0 = wid * rows_per_tile
    n_chunks = rows_per_tile // IDX_CHUNK

    def idx_fetch(slot, base):
      pltpu.async_copy(src_hbm.at[pl.ds(base, IDX_CHUNK)], src_v.at[slot],
                       isem[slot])
      pltpu.async_copy(dst_hbm.at[pl.ds(base, IDX_CHUNK)], dst_v.at[slot],
                       isem[slot])

    def idx_drain(slot):
      pltpu.make_async_copy(src_hbm.at[pl.ds(row0, IDX_CHUNK)],
                            src_v.at[slot], isem[slot]).wait()
      pltpu.make_async_copy(dst_hbm.at[pl.ds(row0, IDX_CHUNK)],
                            dst_v.at[slot], isem[slot]).wait()

    idx_fetch(0, row0)
    idx_fetch(1, row0 + IDX_CHUNK)

    def pair_body(p, _):
      pre0 = row0 + jnp.minimum(2 * p + 2, n_chunks - 1) * IDX_CHUNK
      pre1 = row0 + jnp.minimum(2 * p + 3, n_chunks - 1) * IDX_CHUNK
      idx_drain(0)
      total = PAIR
      gd = [None] * (total + 1)
      sd = [None] * total
      gd[0] = pltpu.async_copy(table_hbm.at[src_v.at[0].at[0]], rows_v.at[0],
                               gsem[0])
      for t in range(total):
        b = t % 2
        nb = 1 - b
        if t + 1 < total:
          if t + 1 == IDX_CHUNK:
            idx_drain(1)
          if t >= 1:
            sd[t - 1].wait()
          sl, g = (t + 1) // IDX_CHUNK, (t + 1) % IDX_CHUNK
          gd[t + 1] = pltpu.async_copy(table_hbm.at[src_v.at[sl].at[g]],
                                       rows_v.at[nb], gsem[nb])
        gd[t].wait()
        sl, g = t // IDX_CHUNK, t % IDX_CHUNK
        sd[t] = pltpu.async_copy(rows_v.at[b], acc_sh.at[dst_v.at[sl].at[g]],
                                 ssem[b], add=True)
        if t == IDX_CHUNK:
          idx_fetch(0, pre0)
      sd[total - 2].wait()
      sd[total - 1].wait()
      idx_fetch(1, pre1)
      return 0
    lax.fori_loop(0, n_chunks // 2, pair_body, 0)
    idx_drain(0)
    idx_drain(1)
    plsc.subcore_barrier()

    def wb(k, _):
      base = s * per + k * STREAM
      pltpu.sync_copy(acc_sh.at[pl.ds(base, STREAM)], rows_v.at[0])
      pltpu.sync_copy(rows_v.at[0], sum_out.at[c, pl.ds(base, STREAM)])
      return 0
    lax.fori_loop(0, per // STREAM, wb, 0)

  return agg(table, src2d, dst2d)


def _dotT(a, w):
  return lax.dot_general(a, w, (((1,), (1,)), ((), ())),
                         preferred_element_type=jnp.float32)


def _tc_layer1(sums, degs, x, Wl, b, Wr):
  d_out = Wl.shape[0]

  def body(sums_ref, degs_ref, x_ref, wl_ref, b_ref, wr_ref, o_ref, inv_ref):
    i = pl.program_id(0)
    agg = sums_ref[0] + sums_ref[1]
    inv = 1.0 / jnp.maximum(degs_ref[0] + degs_ref[1], 1.0)
    inv_ref[...] = inv
    mean = agg * inv
    h = _dotT(mean, wl_ref[...]) + _dotT(x_ref[...], wr_ref[...]) + b_ref[...]
    h = jnp.maximum(h, 0.0)
    rows = i * BLK + lax.broadcasted_iota(jnp.int32, (BLK, 1), 0)
    o_ref[...] = jnp.where(rows < N_NODES, h, 0.0)

  return pl.pallas_call(
      body,
      grid=(N_PAD // BLK,),
      in_specs=[
          pl.BlockSpec((N_SC, BLK, D), lambda i: (0, i, 0)),
          pl.BlockSpec((N_SC, BLK, D), lambda i: (0, i, 0)),
          pl.BlockSpec((BLK, D), lambda i: (i, 0)),
          pl.BlockSpec(Wl.shape, lambda i: (0, 0)),
          pl.BlockSpec((1, d_out), lambda i: (0, 0)),
          pl.BlockSpec(Wr.shape, lambda i: (0, 0)),
      ],
      out_specs=[
          pl.BlockSpec((BLK, d_out), lambda i: (i, 0)),
          pl.BlockSpec((BLK, D), lambda i: (i, 0)),
      ],
      out_shape=[
          jax.ShapeDtypeStruct((N_PAD, d_out), jnp.float32),
          jax.ShapeDtypeStruct((N_PAD, D), jnp.float32),
      ],
  )(sums, degs, x, Wl, b.reshape(1, -1), Wr)


def _tc_layer2(sums, inv, x, Wl, b, Wr):
  d_out = Wl.shape[0]

  def body(sums_ref, inv_ref, x_ref, wl_ref, b_ref, wr_ref, o_ref):
    i = pl.program_id(0)
    mean = (sums_ref[0] + sums_ref[1]) * inv_ref[...]
    h = _dotT(mean, wl_ref[...]) + _dotT(x_ref[...], wr_ref[...]) + b_ref[...]
    h = jnp.maximum(h, 0.0)
    rows = i * BLK + lax.broadcasted_iota(jnp.int32, (BLK, 1), 0)
    o_ref[...] = jnp.where(rows < N_NODES, h, 0.0)

  return pl.pallas_call(
      body,
      grid=(N_PAD // BLK,),
      in_specs=[
          pl.BlockSpec((N_SC, BLK, D), lambda i: (0, i, 0)),
          pl.BlockSpec((BLK, D), lambda i: (i, 0)),
          pl.BlockSpec((BLK, D), lambda i: (i, 0)),
          pl.BlockSpec(Wl.shape, lambda i: (0, 0)),
          pl.BlockSpec((1, d_out), lambda i: (0, 0)),
          pl.BlockSpec(Wr.shape, lambda i: (0, 0)),
      ],
      out_specs=pl.BlockSpec((BLK, d_out), lambda i: (i, 0)),
      out_shape=jax.ShapeDtypeStruct((N_PAD, d_out), jnp.float32),
  )(sums, inv, x, Wl, b.reshape(1, -1), Wr)


def _tc_layer3_fc(sums, inv, x, Wl, b, Wr, Wfc_pad, bfc_pad):
  d_hid = Wl.shape[0]

  def body(sums_ref, inv_ref, x_ref, wl_ref, b_ref, wr_ref, wfc_ref,
           bfc_ref, o_ref):
    mean = (sums_ref[0] + sums_ref[1]) * inv_ref[...]
    h = _dotT(mean, wl_ref[...]) + _dotT(x_ref[...], wr_ref[...]) + b_ref[...]
    h = jnp.maximum(h, 0.0)
    o_ref[...] = _dotT(h, wfc_ref[...]) + bfc_ref[...]

  return pl.pallas_call(
      body,
      grid=(N_PAD // BLK,),
      in_specs=[
          pl.BlockSpec((N_SC, BLK, D), lambda i: (0, i, 0)),
          pl.BlockSpec((BLK, D), lambda i: (i, 0)),
          pl.BlockSpec((BLK, D), lambda i: (i, 0)),
          pl.BlockSpec(Wl.shape, lambda i: (0, 0)),
          pl.BlockSpec((1, d_hid), lambda i: (0, 0)),
          pl.BlockSpec(Wr.shape, lambda i: (0, 0)),
          pl.BlockSpec(Wfc_pad.shape, lambda i: (0, 0)),
          pl.BlockSpec((1, 128), lambda i: (0, 0)),
      ],
      out_specs=pl.BlockSpec((BLK, 128), lambda i: (i, 0)),
      out_shape=jax.ShapeDtypeStruct((N_PAD, 128), jnp.float32),
  )(sums, inv, x, Wl, b.reshape(1, -1), Wr, Wfc_pad, bfc_pad.reshape(1, -1))


def kernel(x, edge_index, W1l, b1, W1r, W2l, b2, W2r, W3l, b3, W3r, Wfc, bfc):
  src = edge_index[0].astype(jnp.int32)
  dst = edge_index[1].astype(jnp.int32)
  n_edges = src.shape[0]

  chunk = N_TILES * STREAM * PAIR
  rows_per_tile = -(-n_edges // chunk) * PAIR
  e_pad = rows_per_tile * N_TILES * STREAM
  n_fill = e_pad - n_edges
  fill_src = jnp.arange(n_fill, dtype=jnp.int32) % N_NODES
  fill_dst = (N_NODES
              + jnp.arange(n_fill, dtype=jnp.int32) % (N_PAD - N_NODES))
  src2d = jnp.concatenate([src, fill_src]).reshape(-1, STREAM)
  dst2d = jnp.concatenate([dst, fill_dst]).reshape(-1, STREAM)

  Wfc_pad = jnp.zeros((128, Wfc.shape[1]), jnp.float32).at[:Wfc.shape[0]].set(Wfc)
  bfc_pad = jnp.zeros((128,), jnp.float32).at[:bfc.shape[0]].set(bfc)

  s1, degs = _sc_aggregate(x, src2d, dst2d, rows_per_tile, with_deg=True)
  h1, inv = _tc_layer1(s1, degs, x, W1l, b1, W1r)
  s2 = _sc_aggregate(h1, src2d, dst2d, rows_per_tile)
  h2 = _tc_layer2(s2, inv, h1, W2l, b2, W2r)
  s3 = _sc_aggregate(h2, src2d, dst2d, rows_per_tile)
  out = _tc_layer3_fc(s3, inv, h2, W3l, b3, W3r, Wfc_pad, bfc_pad)
  return out[:N_NODES, :Wfc.shape[0]]


# speedup vs baseline: 10.1341x; 1.0058x over previous
import functools

import jax
import jax.numpy as jnp
from jax import lax
from jax.experimental import pallas as pl
from jax.experimental.pallas import tpu as pltpu
from jax.experimental.pallas import tpu_sc as plsc

N_NODES = 10000
N_PAD = 10240
D = 128
STREAM = 128
N_SC = 2
N_SUB = 16
N_TILES = N_SC * N_SUB
IDX_CHUNK = 8
PAIR = 2 * IDX_CHUNK
BLK = 256


def _zero_buf(buf, rows, cols):
  def zrow(i, _):
    def zcol(j, _):
      buf[i, pl.ds(j * 16, 16)] = jnp.zeros((16,), jnp.float32)
      return 0
    lax.fori_loop(0, cols // 16, zcol, 0)
    return 0
  lax.fori_loop(0, rows, zrow, 0)


def _ones_buf(buf, rows, cols):
  def orow(i, _):
    def ocol(j, _):
      buf[i, pl.ds(j * 16, 16)] = jnp.ones((16,), jnp.float32)
      return 0
    lax.fori_loop(0, cols // 16, ocol, 0)
    return 0
  lax.fori_loop(0, rows, orow, 0)


def _sc_aggregate(table, src2d, dst2d, rows_per_tile, with_deg=False):
  mesh = plsc.VectorSubcoreMesh(core_axis_name="c", subcore_axis_name="s")
  per = N_PAD // N_SUB
  out_type = jax.ShapeDtypeStruct((N_SC, N_PAD, D), jnp.float32)

  @functools.partial(
      pl.kernel,
      out_type=[out_type, out_type] if with_deg else out_type,
      mesh=mesh,
      scratch_types=[
          pltpu.VMEM((2, IDX_CHUNK, STREAM), jnp.int32),
          pltpu.VMEM((2, IDX_CHUNK, STREAM), jnp.int32),
          pltpu.VMEM((2, STREAM, D), jnp.float32),
          pltpu.VMEM_SHARED((N_PAD, D), jnp.float32),
          pltpu.SemaphoreType.DMA,
          pltpu.SemaphoreType.DMA,
          pltpu.SemaphoreType.DMA,
          pltpu.SemaphoreType.DMA,
          pltpu.SemaphoreType.DMA,
          pltpu.SemaphoreType.DMA,
      ],
  )
  def agg(table_hbm, src_hbm, dst_hbm, *out_and_scratch):
    if with_deg:
      (sum_out, deg_out, src_v, dst_v, rows_v, acc_sh,
       gsem0, gsem1, ssem0, ssem1, isem0, isem1) = out_and_scratch
    else:
      (sum_out, src_v, dst_v, rows_v, acc_sh,
       gsem0, gsem1, ssem0, ssem1, isem0, isem1) = out_and_scratch
    c = lax.axis_index("c")
    s = lax.axis_index("s")
    wid = c * N_SUB + s
    gsem = (gsem0, gsem1)
    ssem = (ssem0, ssem1)
    isem = (isem0, isem1)
    row0 = wid * rows_per_tile

    _zero_buf(rows_v.at[0], STREAM, D)
    def zshared(k, _):
      pltpu.sync_copy(rows_v.at[0],
                      acc_sh.at[pl.ds(s * per + k * STREAM, STREAM)])
      return 0

    if with_deg:
      _ones_buf(rows_v.at[1], STREAM, D)
      lax.fori_loop(0, per // STREAM, zshared, 0)
      plsc.subcore_barrier()
      def deg_chunk(k, _):
        base = row0 + k * IDX_CHUNK
        pltpu.sync_copy(dst_hbm.at[pl.ds(base, IDX_CHUNK)], dst_v.at[0])
        sds = [pltpu.async_copy(rows_v.at[1], acc_sh.at[dst_v.at[0].at[g]],
                                ssem0, add=True)
               for g in range(IDX_CHUNK)]
        for d in sds:
          d.wait()
        return 0
      lax.fori_loop(0, rows_per_tile // IDX_CHUNK, deg_chunk, 0)
      plsc.subcore_barrier()
      def deg_wb(k, _):
        base = s * per + k * STREAM
        pltpu.sync_copy(acc_sh.at[pl.ds(base, STREAM)],
                        deg_out.at[c, pl.ds(base, STREAM)])
        return 0
      lax.fori_loop(0, per // STREAM, deg_wb, 0)

    lax.fori_loop(0, per // STREAM, zshared, 0)
    plsc.subcore_barrier()

    row0 = wid * rows_per_tile
    n_chunks = rows_per_tile // IDX_CHUNK

    def idx_fetch(slot, base):
      pltpu.async_copy(src_hbm.at[pl.ds(base, IDX_CHUNK)], src_v.at[slot],
                       isem[slot])
      pltpu.async_copy(dst_hbm.at[pl.ds(base, IDX_CHUNK)], dst_v.at[slot],
                       isem[slot])

    def idx_drain(slot):
      pltpu.make_async_copy(src_hbm.at[pl.ds(row0, IDX_CHUNK)],
                            src_v.at[slot], isem[slot]).wait()
      pltpu.make_async_copy(dst_hbm.at[pl.ds(row0, IDX_CHUNK)],
                            dst_v.at[slot], isem[slot]).wait()

    idx_fetch(0, row0)
    idx_fetch(1, row0 + IDX_CHUNK)

    def pair_body(p, _):
      pre0 = row0 + jnp.minimum(2 * p + 2, n_chunks - 1) * IDX_CHUNK
      pre1 = row0 + jnp.minimum(2 * p + 3, n_chunks - 1) * IDX_CHUNK
      idx_drain(0)
      total = PAIR
      gd = [None] * (total + 1)
      sd = [None] * total
      gd[0] = pltpu.async_copy(table_hbm.at[src_v.at[0].at[0]], rows_v.at[0],
                               gsem[0])
      for t in range(total):
        b = t % 2
        nb = 1 - b
        if t + 1 < total:
          if t + 1 == IDX_CHUNK:
            idx_drain(1)
          if t >= 1:
            sd[t - 1].wait()
          sl, g = (t + 1) // IDX_CHUNK, (t + 1) % IDX_CHUNK
          gd[t + 1] = pltpu.async_copy(table_hbm.at[src_v.at[sl].at[g]],
                                       rows_v.at[nb], gsem[nb])
        gd[t].wait()
        sl, g = t // IDX_CHUNK, t % IDX_CHUNK
        sd[t] = pltpu.async_copy(rows_v.at[b], acc_sh.at[dst_v.at[sl].at[g]],
                                 ssem[b], add=True)
        if t == IDX_CHUNK:
          idx_fetch(0, pre0)
      sd[total - 2].wait()
      sd[total - 1].wait()
      idx_fetch(1, pre1)
      return 0
    lax.fori_loop(0, n_chunks // 2, pair_body, 0)
    idx_drain(0)
    idx_drain(1)
    plsc.subcore_barrier()

    def wb(k, _):
      base = s * per + k * STREAM
      pltpu.sync_copy(acc_sh.at[pl.ds(base, STREAM)],
                      sum_out.at[c, pl.ds(base, STREAM)])
      return 0
    lax.fori_loop(0, per // STREAM, wb, 0)

  return agg(table, src2d, dst2d)


def _dotT(a, w):
  return lax.dot_general(a, w, (((1,), (1,)), ((), ())),
                         preferred_element_type=jnp.float32)


def _tc_layer1(sums, degs, x, Wl, b, Wr):
  d_out = Wl.shape[0]

  def body(sums_ref, degs_ref, x_ref, wl_ref, b_ref, wr_ref, o_ref, inv_ref):
    i = pl.program_id(0)
    agg = sums_ref[0] + sums_ref[1]
    inv = 1.0 / jnp.maximum(degs_ref[0] + degs_ref[1], 1.0)
    inv_ref[...] = inv
    mean = agg * inv
    h = _dotT(mean, wl_ref[...]) + _dotT(x_ref[...], wr_ref[...]) + b_ref[...]
    h = jnp.maximum(h, 0.0)
    rows = i * BLK + lax.broadcasted_iota(jnp.int32, (BLK, 1), 0)
    o_ref[...] = jnp.where(rows < N_NODES, h, 0.0)

  return pl.pallas_call(
      body,
      grid=(N_PAD // BLK,),
      in_specs=[
          pl.BlockSpec((N_SC, BLK, D), lambda i: (0, i, 0)),
          pl.BlockSpec((N_SC, BLK, D), lambda i: (0, i, 0)),
          pl.BlockSpec((BLK, D), lambda i: (i, 0)),
          pl.BlockSpec(Wl.shape, lambda i: (0, 0)),
          pl.BlockSpec((1, d_out), lambda i: (0, 0)),
          pl.BlockSpec(Wr.shape, lambda i: (0, 0)),
      ],
      out_specs=[
          pl.BlockSpec((BLK, d_out), lambda i: (i, 0)),
          pl.BlockSpec((BLK, D), lambda i: (i, 0)),
      ],
      out_shape=[
          jax.ShapeDtypeStruct((N_PAD, d_out), jnp.float32),
          jax.ShapeDtypeStruct((N_PAD, D), jnp.float32),
      ],
  )(sums, degs, x, Wl, b.reshape(1, -1), Wr)


def _tc_layer2(sums, inv, x, Wl, b, Wr):
  d_out = Wl.shape[0]

  def body(sums_ref, inv_ref, x_ref, wl_ref, b_ref, wr_ref, o_ref):
    i = pl.program_id(0)
    mean = (sums_ref[0] + sums_ref[1]) * inv_ref[...]
    h = _dotT(mean, wl_ref[...]) + _dotT(x_ref[...], wr_ref[...]) + b_ref[...]
    h = jnp.maximum(h, 0.0)
    rows = i * BLK + lax.broadcasted_iota(jnp.int32, (BLK, 1), 0)
    o_ref[...] = jnp.where(rows < N_NODES, h, 0.0)

  return pl.pallas_call(
      body,
      grid=(N_PAD // BLK,),
      in_specs=[
          pl.BlockSpec((N_SC, BLK, D), lambda i: (0, i, 0)),
          pl.BlockSpec((BLK, D), lambda i: (i, 0)),
          pl.BlockSpec((BLK, D), lambda i: (i, 0)),
          pl.BlockSpec(Wl.shape, lambda i: (0, 0)),
          pl.BlockSpec((1, d_out), lambda i: (0, 0)),
          pl.BlockSpec(Wr.shape, lambda i: (0, 0)),
      ],
      out_specs=pl.BlockSpec((BLK, d_out), lambda i: (i, 0)),
      out_shape=jax.ShapeDtypeStruct((N_PAD, d_out), jnp.float32),
  )(sums, inv, x, Wl, b.reshape(1, -1), Wr)


def _tc_layer3_fc(sums, inv, x, Wl, b, Wr, Wfc_pad, bfc_pad):
  d_hid = Wl.shape[0]

  def body(sums_ref, inv_ref, x_ref, wl_ref, b_ref, wr_ref, wfc_ref,
           bfc_ref, o_ref):
    mean = (sums_ref[0] + sums_ref[1]) * inv_ref[...]
    h = _dotT(mean, wl_ref[...]) + _dotT(x_ref[...], wr_ref[...]) + b_ref[...]
    h = jnp.maximum(h, 0.0)
    o_ref[...] = _dotT(h, wfc_ref[...]) + bfc_ref[...]

  return pl.pallas_call(
      body,
      grid=(N_PAD // BLK,),
      in_specs=[
          pl.BlockSpec((N_SC, BLK, D), lambda i: (0, i, 0)),
          pl.BlockSpec((BLK, D), lambda i: (i, 0)),
          pl.BlockSpec((BLK, D), lambda i: (i, 0)),
          pl.BlockSpec(Wl.shape, lambda i: (0, 0)),
          pl.BlockSpec((1, d_hid), lambda i: (0, 0)),
          pl.BlockSpec(Wr.shape, lambda i: (0, 0)),
          pl.BlockSpec(Wfc_pad.shape, lambda i: (0, 0)),
          pl.BlockSpec((1, 128), lambda i: (0, 0)),
      ],
      out_specs=pl.BlockSpec((BLK, 128), lambda i: (i, 0)),
      out_shape=jax.ShapeDtypeStruct((N_PAD, 128), jnp.float32),
  )(sums, inv, x, Wl, b.reshape(1, -1), Wr, Wfc_pad, bfc_pad.reshape(1, -1))


def kernel(x, edge_index, W1l, b1, W1r, W2l, b2, W2r, W3l, b3, W3r, Wfc, bfc):
  src = edge_index[0].astype(jnp.int32)
  dst = edge_index[1].astype(jnp.int32)
  n_edges = src.shape[0]

  chunk = N_TILES * STREAM * PAIR
  rows_per_tile = -(-n_edges // chunk) * PAIR
  e_pad = rows_per_tile * N_TILES * STREAM
  n_fill = e_pad - n_edges
  fill_src = jnp.arange(n_fill, dtype=jnp.int32) % N_NODES
  fill_dst = (N_NODES
              + jnp.arange(n_fill, dtype=jnp.int32) % (N_PAD - N_NODES))
  src2d = jnp.concatenate([src, fill_src]).reshape(-1, STREAM)
  dst2d = jnp.concatenate([dst, fill_dst]).reshape(-1, STREAM)

  Wfc_pad = jnp.zeros((128, Wfc.shape[1]), jnp.float32).at[:Wfc.shape[0]].set(Wfc)
  bfc_pad = jnp.zeros((128,), jnp.float32).at[:bfc.shape[0]].set(bfc)

  s1, degs = _sc_aggregate(x, src2d, dst2d, rows_per_tile, with_deg=True)
  h1, inv = _tc_layer1(s1, degs, x, W1l, b1, W1r)
  s2 = _sc_aggregate(h1, src2d, dst2d, rows_per_tile)
  h2 = _tc_layer2(s2, inv, h1, W2l, b2, W2r)
  s3 = _sc_aggregate(h2, src2d, dst2d, rows_per_tile)
  out = _tc_layer3_fc(s3, inv, h2, W3l, b3, W3r, Wfc_pad, bfc_pad)
  return out[:N_NODES, :Wfc.shape[0]]


# speedup vs baseline: 10.8072x; 1.0664x over previous
import functools

import jax
import jax.numpy as jnp
from jax import lax
from jax.experimental import pallas as pl
from jax.experimental.pallas import tpu as pltpu
from jax.experimental.pallas import tpu_sc as plsc

N_NODES = 10000
N_PAD = 10240
D = 128
STREAM = 128
N_SC = 2
N_SUB = 16
N_TILES = N_SC * N_SUB
IDX_CHUNK = 8
PAIR = 2 * IDX_CHUNK
BLK = 512


def _zero_buf(buf, rows, cols):
  def zrow(i, _):
    def zcol(j, _):
      buf[i, pl.ds(j * 16, 16)] = jnp.zeros((16,), jnp.float32)
      return 0
    lax.fori_loop(0, cols // 16, zcol, 0)
    return 0
  lax.fori_loop(0, rows, zrow, 0)


def _ones_buf(buf, rows, cols):
  def orow(i, _):
    def ocol(j, _):
      buf[i, pl.ds(j * 16, 16)] = jnp.ones((16,), jnp.float32)
      return 0
    lax.fori_loop(0, cols // 16, ocol, 0)
    return 0
  lax.fori_loop(0, rows, orow, 0)


def _sc_aggregate(table, src2d, dst2d, rows_per_tile, with_deg=False):
  mesh = plsc.VectorSubcoreMesh(core_axis_name="c", subcore_axis_name="s")
  per = N_PAD // N_SUB
  out_type = jax.ShapeDtypeStruct((N_SC, N_PAD, D), jnp.float32)

  @functools.partial(
      pl.kernel,
      out_type=[out_type, out_type] if with_deg else out_type,
      mesh=mesh,
      scratch_types=[
          pltpu.VMEM((2, IDX_CHUNK, STREAM), jnp.int32),
          pltpu.VMEM((2, IDX_CHUNK, STREAM), jnp.int32),
          pltpu.VMEM((2, STREAM, D), jnp.float32),
          pltpu.VMEM_SHARED((N_PAD, D), jnp.float32),
          pltpu.SemaphoreType.DMA,
          pltpu.SemaphoreType.DMA,
          pltpu.SemaphoreType.DMA,
          pltpu.SemaphoreType.DMA,
          pltpu.SemaphoreType.DMA,
          pltpu.SemaphoreType.DMA,
      ],
  )
  def agg(table_hbm, src_hbm, dst_hbm, *out_and_scratch):
    if with_deg:
      (sum_out, deg_out, src_v, dst_v, rows_v, acc_sh,
       gsem0, gsem1, ssem0, ssem1, isem0, isem1) = out_and_scratch
    else:
      (sum_out, src_v, dst_v, rows_v, acc_sh,
       gsem0, gsem1, ssem0, ssem1, isem0, isem1) = out_and_scratch
    c = lax.axis_index("c")
    s = lax.axis_index("s")
    wid = c * N_SUB + s
    gsem = (gsem0, gsem1)
    ssem = (ssem0, ssem1)
    isem = (isem0, isem1)
    row0 = wid * rows_per_tile

    _zero_buf(rows_v.at[0], STREAM, D)
    def zshared(k, _):
      pltpu.sync_copy(rows_v.at[0],
                      acc_sh.at[pl.ds(s * per + k * STREAM, STREAM)])
      return 0

    if with_deg:
      _ones_buf(rows_v.at[1], STREAM, D)
      lax.fori_loop(0, per // STREAM, zshared, 0)
      plsc.subcore_barrier()
      def deg_chunk(k, _):
        base = row0 + k * IDX_CHUNK
        pltpu.sync_copy(dst_hbm.at[pl.ds(base, IDX_CHUNK)], dst_v.at[0])
        sds = [pltpu.async_copy(rows_v.at[1], acc_sh.at[dst_v.at[0].at[g]],
                                ssem0, add=True)
               for g in range(IDX_CHUNK)]
        for d in sds:
          d.wait()
        return 0
      lax.fori_loop(0, rows_per_tile // IDX_CHUNK, deg_chunk, 0)
      plsc.subcore_barrier()
      def deg_wb(k, _):
        base = s * per + k * STREAM
        pltpu.sync_copy(acc_sh.at[pl.ds(base, STREAM)],
                        deg_out.at[c, pl.ds(base, STREAM)])
        return 0
      lax.fori_loop(0, per // STREAM, deg_wb, 0)

    lax.fori_loop(0, per // STREAM, zshared, 0)
    plsc.subcore_barrier()

    row0 = wid * rows_per_tile
    n_chunks = rows_per_tile // IDX_CHUNK

    def idx_fetch(slot, base):
      pltpu.async_copy(src_hbm.at[pl.ds(base, IDX_CHUNK)], src_v.at[slot],
                       isem[slot])
      pltpu.async_copy(dst_hbm.at[pl.ds(base, IDX_CHUNK)], dst_v.at[slot],
                       isem[slot])

    def idx_drain(slot):
      pltpu.make_async_copy(src_hbm.at[pl.ds(row0, IDX_CHUNK)],
                            src_v.at[slot], isem[slot]).wait()
      pltpu.make_async_copy(dst_hbm.at[pl.ds(row0, IDX_CHUNK)],
                            dst_v.at[slot], isem[slot]).wait()

    idx_fetch(0, row0)
    idx_fetch(1, row0 + IDX_CHUNK)

    def pair_body(p, _):
      pre0 = row0 + jnp.minimum(2 * p + 2, n_chunks - 1) * IDX_CHUNK
      pre1 = row0 + jnp.minimum(2 * p + 3, n_chunks - 1) * IDX_CHUNK
      idx_drain(0)
      total = PAIR
      gd = [None] * (total + 1)
      sd = [None] * total
      gd[0] = pltpu.async_copy(table_hbm.at[src_v.at[0].at[0]], rows_v.at[0],
                               gsem[0])
      for t in range(total):
        b = t % 2
        nb = 1 - b
        if t + 1 < total:
          if t + 1 == IDX_CHUNK:
            idx_drain(1)
          if t >= 1:
            sd[t - 1].wait()
          sl, g = (t + 1) // IDX_CHUNK, (t + 1) % IDX_CHUNK
          gd[t + 1] = pltpu.async_copy(table_hbm.at[src_v.at[sl].at[g]],
                                       rows_v.at[nb], gsem[nb])
        gd[t].wait()
        sl, g = t // IDX_CHUNK, t % IDX_CHUNK
        sd[t] = pltpu.async_copy(rows_v.at[b], acc_sh.at[dst_v.at[sl].at[g]],
                                 ssem[b], add=True)
        if t == IDX_CHUNK:
          idx_fetch(0, pre0)
      sd[total - 2].wait()
      sd[total - 1].wait()
      idx_fetch(1, pre1)
      return 0
    lax.fori_loop(0, n_chunks // 2, pair_body, 0)
    idx_drain(0)
    idx_drain(1)
    plsc.subcore_barrier()

    def wb(k, _):
      base = s * per + k * STREAM
      pltpu.sync_copy(acc_sh.at[pl.ds(base, STREAM)],
                      sum_out.at[c, pl.ds(base, STREAM)])
      return 0
    lax.fori_loop(0, per // STREAM, wb, 0)

  return agg(table, src2d, dst2d)


def _dotT(a, w):
  return lax.dot_general(a, w, (((1,), (1,)), ((), ())),
                         preferred_element_type=jnp.float32)


def _tc_layer1(sums, degs, x, Wl, b, Wr):
  d_out = Wl.shape[0]

  def body(sums_ref, degs_ref, x_ref, wl_ref, b_ref, wr_ref, o_ref, inv_ref):
    i = pl.program_id(0)
    agg = sums_ref[0] + sums_ref[1]
    inv = 1.0 / jnp.maximum(degs_ref[0] + degs_ref[1], 1.0)
    inv_ref[...] = inv
    mean = agg * inv
    h = _dotT(mean, wl_ref[...]) + _dotT(x_ref[...], wr_ref[...]) + b_ref[...]
    h = jnp.maximum(h, 0.0)
    rows = i * BLK + lax.broadcasted_iota(jnp.int32, (BLK, 1), 0)
    o_ref[...] = jnp.where(rows < N_NODES, h, 0.0)

  return pl.pallas_call(
      body,
      grid=(N_PAD // BLK,),
      in_specs=[
          pl.BlockSpec((N_SC, BLK, D), lambda i: (0, i, 0)),
          pl.BlockSpec((N_SC, BLK, D), lambda i: (0, i, 0)),
          pl.BlockSpec((BLK, D), lambda i: (i, 0)),
          pl.BlockSpec(Wl.shape, lambda i: (0, 0)),
          pl.BlockSpec((1, d_out), lambda i: (0, 0)),
          pl.BlockSpec(Wr.shape, lambda i: (0, 0)),
      ],
      out_specs=[
          pl.BlockSpec((BLK, d_out), lambda i: (i, 0)),
          pl.BlockSpec((BLK, D), lambda i: (i, 0)),
      ],
      out_shape=[
          jax.ShapeDtypeStruct((N_PAD, d_out), jnp.float32),
          jax.ShapeDtypeStruct((N_PAD, D), jnp.float32),
      ],
  )(sums, degs, x, Wl, b.reshape(1, -1), Wr)


def _tc_layer2(sums, inv, x, Wl, b, Wr):
  d_out = Wl.shape[0]

  def body(sums_ref, inv_ref, x_ref, wl_ref, b_ref, wr_ref, o_ref):
    i = pl.program_id(0)
    mean = (sums_ref[0] + sums_ref[1]) * inv_ref[...]
    h = _dotT(mean, wl_ref[...]) + _dotT(x_ref[...], wr_ref[...]) + b_ref[...]
    h = jnp.maximum(h, 0.0)
    rows = i * BLK + lax.broadcasted_iota(jnp.int32, (BLK, 1), 0)
    o_ref[...] = jnp.where(rows < N_NODES, h, 0.0)

  return pl.pallas_call(
      body,
      grid=(N_PAD // BLK,),
      in_specs=[
          pl.BlockSpec((N_SC, BLK, D), lambda i: (0, i, 0)),
          pl.BlockSpec((BLK, D), lambda i: (i, 0)),
          pl.BlockSpec((BLK, D), lambda i: (i, 0)),
          pl.BlockSpec(Wl.shape, lambda i: (0, 0)),
          pl.BlockSpec((1, d_out), lambda i: (0, 0)),
          pl.BlockSpec(Wr.shape, lambda i: (0, 0)),
      ],
      out_specs=pl.BlockSpec((BLK, d_out), lambda i: (i, 0)),
      out_shape=jax.ShapeDtypeStruct((N_PAD, d_out), jnp.float32),
  )(sums, inv, x, Wl, b.reshape(1, -1), Wr)


def _tc_layer3_fc(sums, inv, x, Wl, b, Wr, Wfc_pad, bfc_pad):
  d_hid = Wl.shape[0]

  def body(sums_ref, inv_ref, x_ref, wl_ref, b_ref, wr_ref, wfc_ref,
           bfc_ref, o_ref):
    mean = (sums_ref[0] + sums_ref[1]) * inv_ref[...]
    h = _dotT(mean, wl_ref[...]) + _dotT(x_ref[...], wr_ref[...]) + b_ref[...]
    h = jnp.maximum(h, 0.0)
    o_ref[...] = _dotT(h, wfc_ref[...]) + bfc_ref[...]

  return pl.pallas_call(
      body,
      grid=(N_PAD // BLK,),
      in_specs=[
          pl.BlockSpec((N_SC, BLK, D), lambda i: (0, i, 0)),
          pl.BlockSpec((BLK, D), lambda i: (i, 0)),
          pl.BlockSpec((BLK, D), lambda i: (i, 0)),
          pl.BlockSpec(Wl.shape, lambda i: (0, 0)),
          pl.BlockSpec((1, d_hid), lambda i: (0, 0)),
          pl.BlockSpec(Wr.shape, lambda i: (0, 0)),
          pl.BlockSpec(Wfc_pad.shape, lambda i: (0, 0)),
          pl.BlockSpec((1, 128), lambda i: (0, 0)),
      ],
      out_specs=pl.BlockSpec((BLK, 128), lambda i: (i, 0)),
      out_shape=jax.ShapeDtypeStruct((N_PAD, 128), jnp.float32),
  )(sums, inv, x, Wl, b.reshape(1, -1), Wr, Wfc_pad, bfc_pad.reshape(1, -1))


def kernel(x, edge_index, W1l, b1, W1r, W2l, b2, W2r, W3l, b3, W3r, Wfc, bfc):
  src = edge_index[0].astype(jnp.int32)
  dst = edge_index[1].astype(jnp.int32)
  n_edges = src.shape[0]

  chunk = N_TILES * STREAM * PAIR
  rows_per_tile = -(-n_edges // chunk) * PAIR
  e_pad = rows_per_tile * N_TILES * STREAM
  n_fill = e_pad - n_edges
  fill_src = jnp.arange(n_fill, dtype=jnp.int32) % N_NODES
  fill_dst = (N_NODES
              + jnp.arange(n_fill, dtype=jnp.int32) % (N_PAD - N_NODES))
  src2d = jnp.concatenate([src, fill_src]).reshape(-1, STREAM)
  dst2d = jnp.concatenate([dst, fill_dst]).reshape(-1, STREAM)

  Wfc_pad = jnp.zeros((128, Wfc.shape[1]), jnp.float32).at[:Wfc.shape[0]].set(Wfc)
  bfc_pad = jnp.zeros((128,), jnp.float32).at[:bfc.shape[0]].set(bfc)

  s1, degs = _sc_aggregate(x, src2d, dst2d, rows_per_tile, with_deg=True)
  h1, inv = _tc_layer1(s1, degs, x, W1l, b1, W1r)
  s2 = _sc_aggregate(h1, src2d, dst2d, rows_per_tile)
  h2 = _tc_layer2(s2, inv, h1, W2l, b2, W2r)
  s3 = _sc_aggregate(h2, src2d, dst2d, rows_per_tile)
  out = _tc_layer3_fc(s3, inv, h2, W3l, b3, W3r, Wfc_pad, bfc_pad)
  return out[:N_NODES, :Wfc.shape[0]]


# speedup vs baseline: 11.2012x; 1.0365x over previous
import functools

import jax
import jax.numpy as jnp
from jax import lax
from jax.experimental import pallas as pl
from jax.experimental.pallas import tpu as pltpu
from jax.experimental.pallas import tpu_sc as plsc

N_NODES = 10000
N_PAD = 10240
D = 128
STREAM = 128
N_SC = 2
N_SUB = 16
N_TILES = N_SC * N_SUB
IDX_CHUNK = 8
PAIR = 2 * IDX_CHUNK
BLK = 1024


def _zero_buf(buf, rows, cols):
  def zrow(i, _):
    def zcol(j, _):
      buf[i, pl.ds(j * 16, 16)] = jnp.zeros((16,), jnp.float32)
      return 0
    lax.fori_loop(0, cols // 16, zcol, 0)
    return 0
  lax.fori_loop(0, rows, zrow, 0)


def _ones_buf(buf, rows, cols):
  def orow(i, _):
    def ocol(j, _):
      buf[i, pl.ds(j * 16, 16)] = jnp.ones((16,), jnp.float32)
      return 0
    lax.fori_loop(0, cols // 16, ocol, 0)
    return 0
  lax.fori_loop(0, rows, orow, 0)


def _sc_aggregate(table, src2d, dst2d, rows_per_tile, with_deg=False):
  mesh = plsc.VectorSubcoreMesh(core_axis_name="c", subcore_axis_name="s")
  per = N_PAD // N_SUB
  out_type = jax.ShapeDtypeStruct((N_SC, N_PAD, D), jnp.float32)

  @functools.partial(
      pl.kernel,
      out_type=[out_type, out_type] if with_deg else out_type,
      mesh=mesh,
      scratch_types=[
          pltpu.VMEM((2, IDX_CHUNK, STREAM), jnp.int32),
          pltpu.VMEM((2, IDX_CHUNK, STREAM), jnp.int32),
          pltpu.VMEM((2, STREAM, D), jnp.float32),
          pltpu.VMEM_SHARED((N_PAD, D), jnp.float32),
          pltpu.SemaphoreType.DMA,
          pltpu.SemaphoreType.DMA,
          pltpu.SemaphoreType.DMA,
          pltpu.SemaphoreType.DMA,
          pltpu.SemaphoreType.DMA,
          pltpu.SemaphoreType.DMA,
      ],
  )
  def agg(table_hbm, src_hbm, dst_hbm, *out_and_scratch):
    if with_deg:
      (sum_out, deg_out, src_v, dst_v, rows_v, acc_sh,
       gsem0, gsem1, ssem0, ssem1, isem0, isem1) = out_and_scratch
    else:
      (sum_out, src_v, dst_v, rows_v, acc_sh,
       gsem0, gsem1, ssem0, ssem1, isem0, isem1) = out_and_scratch
    c = lax.axis_index("c")
    s = lax.axis_index("s")
    wid = c * N_SUB + s
    gsem = (gsem0, gsem1)
    ssem = (ssem0, ssem1)
    isem = (isem0, isem1)
    row0 = wid * rows_per_tile

    _zero_buf(rows_v.at[0], STREAM, D)
    def zshared(k, _):
      pltpu.sync_copy(rows_v.at[0],
                      acc_sh.at[pl.ds(s * per + k * STREAM, STREAM)])
      return 0

    if with_deg:
      _ones_buf(rows_v.at[1], STREAM, D)
      lax.fori_loop(0, per // STREAM, zshared, 0)
      plsc.subcore_barrier()
      def deg_chunk(k, _):
        base = row0 + k * IDX_CHUNK
        pltpu.sync_copy(dst_hbm.at[pl.ds(base, IDX_CHUNK)], dst_v.at[0])
        sds = [pltpu.async_copy(rows_v.at[1], acc_sh.at[dst_v.at[0].at[g]],
                                ssem0, add=True)
               for g in range(IDX_CHUNK)]
        for d in sds:
          d.wait()
        return 0
      lax.fori_loop(0, rows_per_tile // IDX_CHUNK, deg_chunk, 0)
      plsc.subcore_barrier()
      def deg_wb(k, _):
        base = s * per + k * STREAM
        pltpu.sync_copy(acc_sh.at[pl.ds(base, STREAM)],
                        deg_out.at[c, pl.ds(base, STREAM)])
        return 0
      lax.fori_loop(0, per // STREAM, deg_wb, 0)

    lax.fori_loop(0, per // STREAM, zshared, 0)
    plsc.subcore_barrier()

    row0 = wid * rows_per_tile
    n_chunks = rows_per_tile // IDX_CHUNK

    def idx_fetch(slot, base):
      pltpu.async_copy(src_hbm.at[pl.ds(base, IDX_CHUNK)], src_v.at[slot],
                       isem[slot])
      pltpu.async_copy(dst_hbm.at[pl.ds(base, IDX_CHUNK)], dst_v.at[slot],
                       isem[slot])

    def idx_drain(slot):
      pltpu.make_async_copy(src_hbm.at[pl.ds(row0, IDX_CHUNK)],
                            src_v.at[slot], isem[slot]).wait()
      pltpu.make_async_copy(dst_hbm.at[pl.ds(row0, IDX_CHUNK)],
                            dst_v.at[slot], isem[slot]).wait()

    idx_fetch(0, row0)
    idx_fetch(1, row0 + IDX_CHUNK)

    def pair_body(p, _):
      pre0 = row0 + jnp.minimum(2 * p + 2, n_chunks - 1) * IDX_CHUNK
      pre1 = row0 + jnp.minimum(2 * p + 3, n_chunks - 1) * IDX_CHUNK
      idx_drain(0)
      total = PAIR
      gd = [None] * (total + 1)
      sd = [None] * total
      gd[0] = pltpu.async_copy(table_hbm.at[src_v.at[0].at[0]], rows_v.at[0],
                               gsem[0])
      for t in range(total):
        b = t % 2
        nb = 1 - b
        if t + 1 < total:
          if t + 1 == IDX_CHUNK:
            idx_drain(1)
          if t >= 1:
            sd[t - 1].wait()
          sl, g = (t + 1) // IDX_CHUNK, (t + 1) % IDX_CHUNK
          gd[t + 1] = pltpu.async_copy(table_hbm.at[src_v.at[sl].at[g]],
                                       rows_v.at[nb], gsem[nb])
        gd[t].wait()
        sl, g = t // IDX_CHUNK, t % IDX_CHUNK
        sd[t] = pltpu.async_copy(rows_v.at[b], acc_sh.at[dst_v.at[sl].at[g]],
                                 ssem[b], add=True)
        if t == IDX_CHUNK:
          idx_fetch(0, pre0)
      sd[total - 2].wait()
      sd[total - 1].wait()
      idx_fetch(1, pre1)
      return 0
    lax.fori_loop(0, n_chunks // 2, pair_body, 0)
    idx_drain(0)
    idx_drain(1)
    plsc.subcore_barrier()

    def wb(k, _):
      base = s * per + k * STREAM
      pltpu.sync_copy(acc_sh.at[pl.ds(base, STREAM)],
                      sum_out.at[c, pl.ds(base, STREAM)])
      return 0
    lax.fori_loop(0, per // STREAM, wb, 0)

  return agg(table, src2d, dst2d)


def _dotT(a, w):
  return lax.dot_general(a, w, (((1,), (1,)), ((), ())),
                         preferred_element_type=jnp.float32)


def _tc_layer1(sums, degs, x, Wl, b, Wr):
  d_out = Wl.shape[0]

  def body(sums_ref, degs_ref, x_ref, wl_ref, b_ref, wr_ref, o_ref, inv_ref):
    i = pl.program_id(0)
    agg = sums_ref[0] + sums_ref[1]
    inv = 1.0 / jnp.maximum(degs_ref[0] + degs_ref[1], 1.0)
    inv_ref[...] = inv
    mean = agg * inv
    h = _dotT(mean, wl_ref[...]) + _dotT(x_ref[...], wr_ref[...]) + b_ref[...]
    h = jnp.maximum(h, 0.0)
    rows = i * BLK + lax.broadcasted_iota(jnp.int32, (BLK, 1), 0)
    o_ref[...] = jnp.where(rows < N_NODES, h, 0.0)

  return pl.pallas_call(
      body,
      grid=(N_PAD // BLK,),
      in_specs=[
          pl.BlockSpec((N_SC, BLK, D), lambda i: (0, i, 0)),
          pl.BlockSpec((N_SC, BLK, D), lambda i: (0, i, 0)),
          pl.BlockSpec((BLK, D), lambda i: (i, 0)),
          pl.BlockSpec(Wl.shape, lambda i: (0, 0)),
          pl.BlockSpec((1, d_out), lambda i: (0, 0)),
          pl.BlockSpec(Wr.shape, lambda i: (0, 0)),
      ],
      out_specs=[
          pl.BlockSpec((BLK, d_out), lambda i: (i, 0)),
          pl.BlockSpec((BLK, D), lambda i: (i, 0)),
      ],
      out_shape=[
          jax.ShapeDtypeStruct((N_PAD, d_out), jnp.float32),
          jax.ShapeDtypeStruct((N_PAD, D), jnp.float32),
      ],
  )(sums, degs, x, Wl, b.reshape(1, -1), Wr)


def _tc_layer2(sums, inv, x, Wl, b, Wr):
  d_out = Wl.shape[0]

  def body(sums_ref, inv_ref, x_ref, wl_ref, b_ref, wr_ref, o_ref):
    i = pl.program_id(0)
    mean = (sums_ref[0] + sums_ref[1]) * inv_ref[...]
    h = _dotT(mean, wl_ref[...]) + _dotT(x_ref[...], wr_ref[...]) + b_ref[...]
    h = jnp.maximum(h, 0.0)
    rows = i * BLK + lax.broadcasted_iota(jnp.int32, (BLK, 1), 0)
    o_ref[...] = jnp.where(rows < N_NODES, h, 0.0)

  return pl.pallas_call(
      body,
      grid=(N_PAD // BLK,),
      in_specs=[
          pl.BlockSpec((N_SC, BLK, D), lambda i: (0, i, 0)),
          pl.BlockSpec((BLK, D), lambda i: (i, 0)),
          pl.BlockSpec((BLK, D), lambda i: (i, 0)),
          pl.BlockSpec(Wl.shape, lambda i: (0, 0)),
          pl.BlockSpec((1, d_out), lambda i: (0, 0)),
          pl.BlockSpec(Wr.shape, lambda i: (0, 0)),
      ],
      out_specs=pl.BlockSpec((BLK, d_out), lambda i: (i, 0)),
      out_shape=jax.ShapeDtypeStruct((N_PAD, d_out), jnp.float32),
  )(sums, inv, x, Wl, b.reshape(1, -1), Wr)


def _tc_layer3_fc(sums, inv, x, Wl, b, Wr, Wfc_pad, bfc_pad):
  d_hid = Wl.shape[0]

  def body(sums_ref, inv_ref, x_ref, wl_ref, b_ref, wr_ref, wfc_ref,
           bfc_ref, o_ref):
    mean = (sums_ref[0] + sums_ref[1]) * inv_ref[...]
    h = _dotT(mean, wl_ref[...]) + _dotT(x_ref[...], wr_ref[...]) + b_ref[...]
    h = jnp.maximum(h, 0.0)
    o_ref[...] = _dotT(h, wfc_ref[...]) + bfc_ref[...]

  return pl.pallas_call(
      body,
      grid=(N_PAD // BLK,),
      in_specs=[
          pl.BlockSpec((N_SC, BLK, D), lambda i: (0, i, 0)),
          pl.BlockSpec((BLK, D), lambda i: (i, 0)),
          pl.BlockSpec((BLK, D), lambda i: (i, 0)),
          pl.BlockSpec(Wl.shape, lambda i: (0, 0)),
          pl.BlockSpec((1, d_hid), lambda i: (0, 0)),
          pl.BlockSpec(Wr.shape, lambda i: (0, 0)),
          pl.BlockSpec(Wfc_pad.shape, lambda i: (0, 0)),
          pl.BlockSpec((1, 128), lambda i: (0, 0)),
      ],
      out_specs=pl.BlockSpec((BLK, 128), lambda i: (i, 0)),
      out_shape=jax.ShapeDtypeStruct((N_PAD, 128), jnp.float32),
  )(sums, inv, x, Wl, b.reshape(1, -1), Wr, Wfc_pad, bfc_pad.reshape(1, -1))


def kernel(x, edge_index, W1l, b1, W1r, W2l, b2, W2r, W3l, b3, W3r, Wfc, bfc):
  src = edge_index[0].astype(jnp.int32)
  dst = edge_index[1].astype(jnp.int32)
  n_edges = src.shape[0]

  chunk = N_TILES * STREAM * PAIR
  rows_per_tile = -(-n_edges // chunk) * PAIR
  e_pad = rows_per_tile * N_TILES * STREAM
  n_fill = e_pad - n_edges
  fill_src = jnp.arange(n_fill, dtype=jnp.int32) % N_NODES
  fill_dst = (N_NODES
              + jnp.arange(n_fill, dtype=jnp.int32) % (N_PAD - N_NODES))
  src2d = jnp.concatenate([src, fill_src]).reshape(-1, STREAM)
  dst2d = jnp.concatenate([dst, fill_dst]).reshape(-1, STREAM)

  Wfc_pad = jnp.zeros((128, Wfc.shape[1]), jnp.float32).at[:Wfc.shape[0]].set(Wfc)
  bfc_pad = jnp.zeros((128,), jnp.float32).at[:bfc.shape[0]].set(bfc)

  s1, degs = _sc_aggregate(x, src2d, dst2d, rows_per_tile, with_deg=True)
  h1, inv = _tc_layer1(s1, degs, x, W1l, b1, W1r)
  s2 = _sc_aggregate(h1, src2d, dst2d, rows_per_tile)
  h2 = _tc_layer2(s2, inv, h1, W2l, b2, W2r)
  s3 = _sc_aggregate(h2, src2d, dst2d, rows_per_tile)
  out = _tc_layer3_fc(s3, inv, h2, W3l, b3, W3r, Wfc_pad, bfc_pad)
  return out[:N_NODES, :Wfc.shape[0]]


# speedup vs baseline: 11.3642x; 1.0146x over previous
import functools

import jax
import jax.numpy as jnp
from jax import lax
from jax.experimental import pallas as pl
from jax.experimental.pallas import tpu as pltpu
from jax.experimental.pallas import tpu_sc as plsc

N_NODES = 10000
N_PAD = 10240
D = 128
STREAM = 128
N_SC = 2
N_SUB = 16
N_TILES = N_SC * N_SUB
IDX_CHUNK = 8
PAIR = 2 * IDX_CHUNK
BLK = 2048


def _zero_buf(buf, rows, cols):
  def zrow(i, _):
    def zcol(j, _):
      buf[i, pl.ds(j * 16, 16)] = jnp.zeros((16,), jnp.float32)
      return 0
    lax.fori_loop(0, cols // 16, zcol, 0)
    return 0
  lax.fori_loop(0, rows, zrow, 0)


def _ones_buf(buf, rows, cols):
  def orow(i, _):
    def ocol(j, _):
      buf[i, pl.ds(j * 16, 16)] = jnp.ones((16,), jnp.float32)
      return 0
    lax.fori_loop(0, cols // 16, ocol, 0)
    return 0
  lax.fori_loop(0, rows, orow, 0)


def _sc_aggregate(table, src2d, dst2d, rows_per_tile, with_deg=False):
  mesh = plsc.VectorSubcoreMesh(core_axis_name="c", subcore_axis_name="s")
  per = N_PAD // N_SUB
  out_type = jax.ShapeDtypeStruct((N_SC, N_PAD, D), jnp.float32)

  @functools.partial(
      pl.kernel,
      out_type=[out_type, out_type] if with_deg else out_type,
      mesh=mesh,
      scratch_types=[
          pltpu.VMEM((2, IDX_CHUNK, STREAM), jnp.int32),
          pltpu.VMEM((2, IDX_CHUNK, STREAM), jnp.int32),
          pltpu.VMEM((2, STREAM, D), jnp.float32),
          pltpu.VMEM_SHARED((N_PAD, D), jnp.float32),
          pltpu.SemaphoreType.DMA,
          pltpu.SemaphoreType.DMA,
          pltpu.SemaphoreType.DMA,
          pltpu.SemaphoreType.DMA,
          pltpu.SemaphoreType.DMA,
          pltpu.SemaphoreType.DMA,
      ],
  )
  def agg(table_hbm, src_hbm, dst_hbm, *out_and_scratch):
    if with_deg:
      (sum_out, deg_out, src_v, dst_v, rows_v, acc_sh,
       gsem0, gsem1, ssem0, ssem1, isem0, isem1) = out_and_scratch
    else:
      (sum_out, src_v, dst_v, rows_v, acc_sh,
       gsem0, gsem1, ssem0, ssem1, isem0, isem1) = out_and_scratch
    c = lax.axis_index("c")
    s = lax.axis_index("s")
    wid = c * N_SUB + s
    gsem = (gsem0, gsem1)
    ssem = (ssem0, ssem1)
    isem = (isem0, isem1)
    row0 = wid * rows_per_tile

    _zero_buf(rows_v.at[0], STREAM, D)
    def zshared(k, _):
      pltpu.sync_copy(rows_v.at[0],
                      acc_sh.at[pl.ds(s * per + k * STREAM, STREAM)])
      return 0

    if with_deg:
      _ones_buf(rows_v.at[1], STREAM, D)
      lax.fori_loop(0, per // STREAM, zshared, 0)
      plsc.subcore_barrier()
      def deg_chunk(k, _):
        base = row0 + k * IDX_CHUNK
        pltpu.sync_copy(dst_hbm.at[pl.ds(base, IDX_CHUNK)], dst_v.at[0])
        sds = [pltpu.async_copy(rows_v.at[1], acc_sh.at[dst_v.at[0].at[g]],
                                ssem0, add=True)
               for g in range(IDX_CHUNK)]
        for d in sds:
          d.wait()
        return 0
      lax.fori_loop(0, rows_per_tile // IDX_CHUNK, deg_chunk, 0)
      plsc.subcore_barrier()
      def deg_wb(k, _):
        base = s * per + k * STREAM
        pltpu.sync_copy(acc_sh.at[pl.ds(base, STREAM)],
                        deg_out.at[c, pl.ds(base, STREAM)])
        return 0
      lax.fori_loop(0, per // STREAM, deg_wb, 0)

    lax.fori_loop(0, per // STREAM, zshared, 0)
    plsc.subcore_barrier()

    row0 = wid * rows_per_tile
    n_chunks = rows_per_tile // IDX_CHUNK

    def idx_fetch(slot, base):
      pltpu.async_copy(src_hbm.at[pl.ds(base, IDX_CHUNK)], src_v.at[slot],
                       isem[slot])
      pltpu.async_copy(dst_hbm.at[pl.ds(base, IDX_CHUNK)], dst_v.at[slot],
                       isem[slot])

    def idx_drain(slot):
      pltpu.make_async_copy(src_hbm.at[pl.ds(row0, IDX_CHUNK)],
                            src_v.at[slot], isem[slot]).wait()
      pltpu.make_async_copy(dst_hbm.at[pl.ds(row0, IDX_CHUNK)],
                            dst_v.at[slot], isem[slot]).wait()

    idx_fetch(0, row0)
    idx_fetch(1, row0 + IDX_CHUNK)

    def pair_body(p, _):
      pre0 = row0 + jnp.minimum(2 * p + 2, n_chunks - 1) * IDX_CHUNK
      pre1 = row0 + jnp.minimum(2 * p + 3, n_chunks - 1) * IDX_CHUNK
      idx_drain(0)
      total = PAIR
      gd = [None] * (total + 1)
      sd = [None] * total
      gd[0] = pltpu.async_copy(table_hbm.at[src_v.at[0].at[0]], rows_v.at[0],
                               gsem[0])
      for t in range(total):
        b = t % 2
        nb = 1 - b
        if t + 1 < total:
          if t + 1 == IDX_CHUNK:
            idx_drain(1)
          if t >= 1:
            sd[t - 1].wait()
          sl, g = (t + 1) // IDX_CHUNK, (t + 1) % IDX_CHUNK
          gd[t + 1] = pltpu.async_copy(table_hbm.at[src_v.at[sl].at[g]],
                                       rows_v.at[nb], gsem[nb])
        gd[t].wait()
        sl, g = t // IDX_CHUNK, t % IDX_CHUNK
        sd[t] = pltpu.async_copy(rows_v.at[b], acc_sh.at[dst_v.at[sl].at[g]],
                                 ssem[b], add=True)
        if t == IDX_CHUNK:
          idx_fetch(0, pre0)
      sd[total - 2].wait()
      sd[total - 1].wait()
      idx_fetch(1, pre1)
      return 0
    lax.fori_loop(0, n_chunks // 2, pair_body, 0)
    idx_drain(0)
    idx_drain(1)
    plsc.subcore_barrier()

    def wb(k, _):
      base = s * per + k * STREAM
      pltpu.sync_copy(acc_sh.at[pl.ds(base, STREAM)],
                      sum_out.at[c, pl.ds(base, STREAM)])
      return 0
    lax.fori_loop(0, per // STREAM, wb, 0)

  return agg(table, src2d, dst2d)


def _dotT(a, w):
  return lax.dot_general(a, w, (((1,), (1,)), ((), ())),
                         preferred_element_type=jnp.float32)


def _tc_layer1(sums, degs, x, Wl, b, Wr):
  d_out = Wl.shape[0]

  def body(sums_ref, degs_ref, x_ref, wl_ref, b_ref, wr_ref, o_ref, inv_ref):
    i = pl.program_id(0)
    agg = sums_ref[0] + sums_ref[1]
    inv = 1.0 / jnp.maximum(degs_ref[0] + degs_ref[1], 1.0)
    inv_ref[...] = inv
    mean = agg * inv
    h = _dotT(mean, wl_ref[...]) + _dotT(x_ref[...], wr_ref[...]) + b_ref[...]
    h = jnp.maximum(h, 0.0)
    rows = i * BLK + lax.broadcasted_iota(jnp.int32, (BLK, 1), 0)
    o_ref[...] = jnp.where(rows < N_NODES, h, 0.0)

  return pl.pallas_call(
      body,
      grid=(N_PAD // BLK,),
      in_specs=[
          pl.BlockSpec((N_SC, BLK, D), lambda i: (0, i, 0)),
          pl.BlockSpec((N_SC, BLK, D), lambda i: (0, i, 0)),
          pl.BlockSpec((BLK, D), lambda i: (i, 0)),
          pl.BlockSpec(Wl.shape, lambda i: (0, 0)),
          pl.BlockSpec((1, d_out), lambda i: (0, 0)),
          pl.BlockSpec(Wr.shape, lambda i: (0, 0)),
      ],
      out_specs=[
          pl.BlockSpec((BLK, d_out), lambda i: (i, 0)),
          pl.BlockSpec((BLK, D), lambda i: (i, 0)),
      ],
      out_shape=[
          jax.ShapeDtypeStruct((N_PAD, d_out), jnp.float32),
          jax.ShapeDtypeStruct((N_PAD, D), jnp.float32),
      ],
  )(sums, degs, x, Wl, b.reshape(1, -1), Wr)


def _tc_layer2(sums, inv, x, Wl, b, Wr):
  d_out = Wl.shape[0]

  def body(sums_ref, inv_ref, x_ref, wl_ref, b_ref, wr_ref, o_ref):
    i = pl.program_id(0)
    mean = (sums_ref[0] + sums_ref[1]) * inv_ref[...]
    h = _dotT(mean, wl_ref[...]) + _dotT(x_ref[...], wr_ref[...]) + b_ref[...]
    h = jnp.maximum(h, 0.0)
    rows = i * BLK + lax.broadcasted_iota(jnp.int32, (BLK, 1), 0)
    o_ref[...] = jnp.where(rows < N_NODES, h, 0.0)

  return pl.pallas_call(
      body,
      grid=(N_PAD // BLK,),
      in_specs=[
          pl.BlockSpec((N_SC, BLK, D), lambda i: (0, i, 0)),
          pl.BlockSpec((BLK, D), lambda i: (i, 0)),
          pl.BlockSpec((BLK, D), lambda i: (i, 0)),
          pl.BlockSpec(Wl.shape, lambda i: (0, 0)),
          pl.BlockSpec((1, d_out), lambda i: (0, 0)),
          pl.BlockSpec(Wr.shape, lambda i: (0, 0)),
      ],
      out_specs=pl.BlockSpec((BLK, d_out), lambda i: (i, 0)),
      out_shape=jax.ShapeDtypeStruct((N_PAD, d_out), jnp.float32),
  )(sums, inv, x, Wl, b.reshape(1, -1), Wr)


def _tc_layer3_fc(sums, inv, x, Wl, b, Wr, Wfc_pad, bfc_pad):
  d_hid = Wl.shape[0]

  def body(sums_ref, inv_ref, x_ref, wl_ref, b_ref, wr_ref, wfc_ref,
           bfc_ref, o_ref):
    mean = (sums_ref[0] + sums_ref[1]) * inv_ref[...]
    h = _dotT(mean, wl_ref[...]) + _dotT(x_ref[...], wr_ref[...]) + b_ref[...]
    h = jnp.maximum(h, 0.0)
    o_ref[...] = _dotT(h, wfc_ref[...]) + bfc_ref[...]

  return pl.pallas_call(
      body,
      grid=(N_PAD // BLK,),
      in_specs=[
          pl.BlockSpec((N_SC, BLK, D), lambda i: (0, i, 0)),
          pl.BlockSpec((BLK, D), lambda i: (i, 0)),
          pl.BlockSpec((BLK, D), lambda i: (i, 0)),
          pl.BlockSpec(Wl.shape, lambda i: (0, 0)),
          pl.BlockSpec((1, d_hid), lambda i: (0, 0)),
          pl.BlockSpec(Wr.shape, lambda i: (0, 0)),
          pl.BlockSpec(Wfc_pad.shape, lambda i: (0, 0)),
          pl.BlockSpec((1, 128), lambda i: (0, 0)),
      ],
      out_specs=pl.BlockSpec((BLK, 128), lambda i: (i, 0)),
      out_shape=jax.ShapeDtypeStruct((N_PAD, 128), jnp.float32),
  )(sums, inv, x, Wl, b.reshape(1, -1), Wr, Wfc_pad, bfc_pad.reshape(1, -1))


def kernel(x, edge_index, W1l, b1, W1r, W2l, b2, W2r, W3l, b3, W3r, Wfc, bfc):
  src = edge_index[0].astype(jnp.int32)
  dst = edge_index[1].astype(jnp.int32)
  n_edges = src.shape[0]

  chunk = N_TILES * STREAM * PAIR
  rows_per_tile = -(-n_edges // chunk) * PAIR
  e_pad = rows_per_tile * N_TILES * STREAM
  n_fill = e_pad - n_edges
  fill_src = jnp.arange(n_fill, dtype=jnp.int32) % N_NODES
  fill_dst = (N_NODES
              + jnp.arange(n_fill, dtype=jnp.int32) % (N_PAD - N_NODES))
  src2d = jnp.concatenate([src, fill_src]).reshape(-1, STREAM)
  dst2d = jnp.concatenate([dst, fill_dst]).reshape(-1, STREAM)

  Wfc_pad = jnp.zeros((128, Wfc.shape[1]), jnp.float32).at[:Wfc.shape[0]].set(Wfc)
  bfc_pad = jnp.zeros((128,), jnp.float32).at[:bfc.shape[0]].set(bfc)

  s1, degs = _sc_aggregate(x, src2d, dst2d, rows_per_tile, with_deg=True)
  h1, inv = _tc_layer1(s1, degs, x, W1l, b1, W1r)
  s2 = _sc_aggregate(h1, src2d, dst2d, rows_per_tile)
  h2 = _tc_layer2(s2, inv, h1, W2l, b2, W2r)
  s3 = _sc_aggregate(h2, src2d, dst2d, rows_per_tile)
  out = _tc_layer3_fc(s3, inv, h2, W3l, b3, W3r, Wfc_pad, bfc_pad)
  return out[:N_NODES, :Wfc.shape[0]]


# speedup vs baseline: 11.3955x; 1.0027x over previous
import functools

import jax
import jax.numpy as jnp
from jax import lax
from jax.experimental import pallas as pl
from jax.experimental.pallas import tpu as pltpu
from jax.experimental.pallas import tpu_sc as plsc

N_NODES = 10000
N_PAD = 10240
D = 128
STREAM = 128
N_SC = 2
N_SUB = 16
N_TILES = N_SC * N_SUB
IDX_CHUNK = 8
PAIR = 2 * IDX_CHUNK
BLK = 5120


def _zero_buf(buf, rows, cols):
  def zrow(i, _):
    def zcol(j, _):
      buf[i, pl.ds(j * 16, 16)] = jnp.zeros((16,), jnp.float32)
      return 0
    lax.fori_loop(0, cols // 16, zcol, 0)
    return 0
  lax.fori_loop(0, rows, zrow, 0)


def _ones_buf(buf, rows, cols):
  def orow(i, _):
    def ocol(j, _):
      buf[i, pl.ds(j * 16, 16)] = jnp.ones((16,), jnp.float32)
      return 0
    lax.fori_loop(0, cols // 16, ocol, 0)
    return 0
  lax.fori_loop(0, rows, orow, 0)


def _sc_aggregate(table, src2d, dst2d, rows_per_tile, with_deg=False):
  mesh = plsc.VectorSubcoreMesh(core_axis_name="c", subcore_axis_name="s")
  per = N_PAD // N_SUB
  out_type = jax.ShapeDtypeStruct((N_SC, N_PAD, D), jnp.float32)

  @functools.partial(
      pl.kernel,
      out_type=[out_type, out_type] if with_deg else out_type,
      mesh=mesh,
      scratch_types=[
          pltpu.VMEM((2, IDX_CHUNK, STREAM), jnp.int32),
          pltpu.VMEM((2, IDX_CHUNK, STREAM), jnp.int32),
          pltpu.VMEM((2, STREAM, D), jnp.float32),
          pltpu.VMEM_SHARED((N_PAD, D), jnp.float32),
          pltpu.SemaphoreType.DMA,
          pltpu.SemaphoreType.DMA,
          pltpu.SemaphoreType.DMA,
          pltpu.SemaphoreType.DMA,
          pltpu.SemaphoreType.DMA,
          pltpu.SemaphoreType.DMA,
      ],
  )
  def agg(table_hbm, src_hbm, dst_hbm, *out_and_scratch):
    if with_deg:
      (sum_out, deg_out, src_v, dst_v, rows_v, acc_sh,
       gsem0, gsem1, ssem0, ssem1, isem0, isem1) = out_and_scratch
    else:
      (sum_out, src_v, dst_v, rows_v, acc_sh,
       gsem0, gsem1, ssem0, ssem1, isem0, isem1) = out_and_scratch
    c = lax.axis_index("c")
    s = lax.axis_index("s")
    wid = c * N_SUB + s
    gsem = (gsem0, gsem1)
    ssem = (ssem0, ssem1)
    isem = (isem0, isem1)
    row0 = wid * rows_per_tile

    _zero_buf(rows_v.at[0], STREAM, D)
    def zshared(k, _):
      pltpu.sync_copy(rows_v.at[0],
                      acc_sh.at[pl.ds(s * per + k * STREAM, STREAM)])
      return 0

    if with_deg:
      _ones_buf(rows_v.at[1], STREAM, D)
      lax.fori_loop(0, per // STREAM, zshared, 0)
      plsc.subcore_barrier()
      def deg_chunk(k, _):
        base = row0 + k * IDX_CHUNK
        pltpu.sync_copy(dst_hbm.at[pl.ds(base, IDX_CHUNK)], dst_v.at[0])
        sds = [pltpu.async_copy(rows_v.at[1], acc_sh.at[dst_v.at[0].at[g]],
                                ssem0, add=True)
               for g in range(IDX_CHUNK)]
        for d in sds:
          d.wait()
        return 0
      lax.fori_loop(0, rows_per_tile // IDX_CHUNK, deg_chunk, 0)
      plsc.subcore_barrier()
      def deg_wb(k, _):
        base = s * per + k * STREAM
        pltpu.sync_copy(acc_sh.at[pl.ds(base, STREAM)],
                        deg_out.at[c, pl.ds(base, STREAM)])
        return 0
      lax.fori_loop(0, per // STREAM, deg_wb, 0)

    lax.fori_loop(0, per // STREAM, zshared, 0)
    plsc.subcore_barrier()

    row0 = wid * rows_per_tile
    n_chunks = rows_per_tile // IDX_CHUNK

    def idx_fetch(slot, base):
      pltpu.async_copy(src_hbm.at[pl.ds(base, IDX_CHUNK)], src_v.at[slot],
                       isem[slot])
      pltpu.async_copy(dst_hbm.at[pl.ds(base, IDX_CHUNK)], dst_v.at[slot],
                       isem[slot])

    def idx_drain(slot):
      pltpu.make_async_copy(src_hbm.at[pl.ds(row0, IDX_CHUNK)],
                            src_v.at[slot], isem[slot]).wait()
      pltpu.make_async_copy(dst_hbm.at[pl.ds(row0, IDX_CHUNK)],
                            dst_v.at[slot], isem[slot]).wait()

    idx_fetch(0, row0)
    idx_fetch(1, row0 + IDX_CHUNK)

    def pair_body(p, _):
      pre0 = row0 + jnp.minimum(2 * p + 2, n_chunks - 1) * IDX_CHUNK
      pre1 = row0 + jnp.minimum(2 * p + 3, n_chunks - 1) * IDX_CHUNK
      idx_drain(0)
      total = PAIR
      gd = [None] * (total + 1)
      sd = [None] * total
      gd[0] = pltpu.async_copy(table_hbm.at[src_v.at[0].at[0]], rows_v.at[0],
                               gsem[0])
      for t in range(total):
        b = t % 2
        nb = 1 - b
        if t + 1 < total:
          if t + 1 == IDX_CHUNK:
            idx_drain(1)
          if t >= 1:
            sd[t - 1].wait()
          sl, g = (t + 1) // IDX_CHUNK, (t + 1) % IDX_CHUNK
          gd[t + 1] = pltpu.async_copy(table_hbm.at[src_v.at[sl].at[g]],
                                       rows_v.at[nb], gsem[nb])
        gd[t].wait()
        sl, g = t // IDX_CHUNK, t % IDX_CHUNK
        sd[t] = pltpu.async_copy(rows_v.at[b], acc_sh.at[dst_v.at[sl].at[g]],
                                 ssem[b], add=True)
        if t == IDX_CHUNK:
          idx_fetch(0, pre0)
      sd[total - 2].wait()
      sd[total - 1].wait()
      idx_fetch(1, pre1)
      return 0
    lax.fori_loop(0, n_chunks // 2, pair_body, 0)
    idx_drain(0)
    idx_drain(1)
    plsc.subcore_barrier()

    def wb(k, _):
      base = s * per + k * STREAM
      pltpu.sync_copy(acc_sh.at[pl.ds(base, STREAM)],
                      sum_out.at[c, pl.ds(base, STREAM)])
      return 0
    lax.fori_loop(0, per // STREAM, wb, 0)

  return agg(table, src2d, dst2d)


def _dotT(a, w):
  return lax.dot_general(a, w, (((1,), (1,)), ((), ())),
                         preferred_element_type=jnp.float32)


def _tc_layer1(sums, degs, x, Wl, b, Wr):
  d_out = Wl.shape[0]

  def body(sums_ref, degs_ref, x_ref, wl_ref, b_ref, wr_ref, o_ref, inv_ref):
    i = pl.program_id(0)
    agg = sums_ref[0] + sums_ref[1]
    inv = 1.0 / jnp.maximum(degs_ref[0] + degs_ref[1], 1.0)
    inv_ref[...] = inv
    mean = agg * inv
    h = _dotT(mean, wl_ref[...]) + _dotT(x_ref[...], wr_ref[...]) + b_ref[...]
    h = jnp.maximum(h, 0.0)
    rows = i * BLK + lax.broadcasted_iota(jnp.int32, (BLK, 1), 0)
    o_ref[...] = jnp.where(rows < N_NODES, h, 0.0)

  return pl.pallas_call(
      body,
      grid=(N_PAD // BLK,),
      in_specs=[
          pl.BlockSpec((N_SC, BLK, D), lambda i: (0, i, 0)),
          pl.BlockSpec((N_SC, BLK, D), lambda i: (0, i, 0)),
          pl.BlockSpec((BLK, D), lambda i: (i, 0)),
          pl.BlockSpec(Wl.shape, lambda i: (0, 0)),
          pl.BlockSpec((1, d_out), lambda i: (0, 0)),
          pl.BlockSpec(Wr.shape, lambda i: (0, 0)),
      ],
      out_specs=[
          pl.BlockSpec((BLK, d_out), lambda i: (i, 0)),
          pl.BlockSpec((BLK, D), lambda i: (i, 0)),
      ],
      out_shape=[
          jax.ShapeDtypeStruct((N_PAD, d_out), jnp.float32),
          jax.ShapeDtypeStruct((N_PAD, D), jnp.float32),
      ],
  )(sums, degs, x, Wl, b.reshape(1, -1), Wr)


def _tc_layer2(sums, inv, x, Wl, b, Wr):
  d_out = Wl.shape[0]

  def body(sums_ref, inv_ref, x_ref, wl_ref, b_ref, wr_ref, o_ref):
    i = pl.program_id(0)
    mean = (sums_ref[0] + sums_ref[1]) * inv_ref[...]
    h = _dotT(mean, wl_ref[...]) + _dotT(x_ref[...], wr_ref[...]) + b_ref[...]
    h = jnp.maximum(h, 0.0)
    rows = i * BLK + lax.broadcasted_iota(jnp.int32, (BLK, 1), 0)
    o_ref[...] = jnp.where(rows < N_NODES, h, 0.0)

  return pl.pallas_call(
      body,
      grid=(N_PAD // BLK,),
      in_specs=[
          pl.BlockSpec((N_SC, BLK, D), lambda i: (0, i, 0)),
          pl.BlockSpec((BLK, D), lambda i: (i, 0)),
          pl.BlockSpec((BLK, D), lambda i: (i, 0)),
          pl.BlockSpec(Wl.shape, lambda i: (0, 0)),
          pl.BlockSpec((1, d_out), lambda i: (0, 0)),
          pl.BlockSpec(Wr.shape, lambda i: (0, 0)),
      ],
      out_specs=pl.BlockSpec((BLK, d_out), lambda i: (i, 0)),
      out_shape=jax.ShapeDtypeStruct((N_PAD, d_out), jnp.float32),
  )(sums, inv, x, Wl, b.reshape(1, -1), Wr)


def _tc_layer3_fc(sums, inv, x, Wl, b, Wr, Wfc_pad, bfc_pad):
  d_hid = Wl.shape[0]

  def body(sums_ref, inv_ref, x_ref, wl_ref, b_ref, wr_ref, wfc_ref,
           bfc_ref, o_ref):
    mean = (sums_ref[0] + sums_ref[1]) * inv_ref[...]
    h = _dotT(mean, wl_ref[...]) + _dotT(x_ref[...], wr_ref[...]) + b_ref[...]
    h = jnp.maximum(h, 0.0)
    o_ref[...] = _dotT(h, wfc_ref[...]) + bfc_ref[...]

  return pl.pallas_call(
      body,
      grid=(N_PAD // BLK,),
      in_specs=[
          pl.BlockSpec((N_SC, BLK, D), lambda i: (0, i, 0)),
          pl.BlockSpec((BLK, D), lambda i: (i, 0)),
          pl.BlockSpec((BLK, D), lambda i: (i, 0)),
          pl.BlockSpec(Wl.shape, lambda i: (0, 0)),
          pl.BlockSpec((1, d_hid), lambda i: (0, 0)),
          pl.BlockSpec(Wr.shape, lambda i: (0, 0)),
          pl.BlockSpec(Wfc_pad.shape, lambda i: (0, 0)),
          pl.BlockSpec((1, 128), lambda i: (0, 0)),
      ],
      out_specs=pl.BlockSpec((BLK, 128), lambda i: (i, 0)),
      out_shape=jax.ShapeDtypeStruct((N_PAD, 128), jnp.float32),
  )(sums, inv, x, Wl, b.reshape(1, -1), Wr, Wfc_pad, bfc_pad.reshape(1, -1))


def kernel(x, edge_index, W1l, b1, W1r, W2l, b2, W2r, W3l, b3, W3r, Wfc, bfc):
  src = edge_index[0].astype(jnp.int32)
  dst = edge_index[1].astype(jnp.int32)
  n_edges = src.shape[0]

  chunk = N_TILES * STREAM * PAIR
  rows_per_tile = -(-n_edges // chunk) * PAIR
  e_pad = rows_per_tile * N_TILES * STREAM
  n_fill = e_pad - n_edges
  fill_src = jnp.arange(n_fill, dtype=jnp.int32) % N_NODES
  fill_dst = (N_NODES
              + jnp.arange(n_fill, dtype=jnp.int32) % (N_PAD - N_NODES))
  src2d = jnp.concatenate([src, fill_src]).reshape(-1, STREAM)
  dst2d = jnp.concatenate([dst, fill_dst]).reshape(-1, STREAM)

  Wfc_pad = jnp.zeros((128, Wfc.shape[1]), jnp.float32).at[:Wfc.shape[0]].set(Wfc)
  bfc_pad = jnp.zeros((128,), jnp.float32).at[:bfc.shape[0]].set(bfc)

  s1, degs = _sc_aggregate(x, src2d, dst2d, rows_per_tile, with_deg=True)
  h1, inv = _tc_layer1(s1, degs, x, W1l, b1, W1r)
  s2 = _sc_aggregate(h1, src2d, dst2d, rows_per_tile)
  h2 = _tc_layer2(s2, inv, h1, W2l, b2, W2r)
  s3 = _sc_aggregate(h2, src2d, dst2d, rows_per_tile)
  out = _tc_layer3_fc(s3, inv, h2, W3l, b3, W3r, Wfc_pad, bfc_pad)
  return out[:N_NODES, :Wfc.shape[0]]
